# baseline (speedup 1.0000x reference)
import functools

import jax
import jax.numpy as jnp
from jax import lax
from jax.experimental import pallas as pl
from jax.experimental.pallas import tpu as pltpu
from jax.experimental.pallas import tpu_sc as plsc

N = 10000
NP = 10240
E = 160000
EP = 163840
ER = EP // 128
RPT = ER // 16
NSUB = 16
NCORE = 2


def _sc_mesh():
    return plsc.VectorSubcoreMesh(core_axis_name="c", subcore_axis_name="s")


_SC_PARAMS = pltpu.CompilerParams(needs_layout_passes=False,
                                  use_tc_tiling_on_sc=False)


@functools.partial(
    pl.kernel,
    out_type=(
        jax.ShapeDtypeStruct((ER, 128), jnp.float32),
        jax.ShapeDtypeStruct((NP,), jnp.float32),
    ),
    mesh=_sc_mesh(),
    compiler_params=_SC_PARAMS,
    scratch_types=[
        pltpu.VMEM_SHARED((NP,), jnp.float32),
        pltpu.VMEM((RPT, 128), jnp.int32),
        pltpu.VMEM((RPT, 128), jnp.float32),
        pltpu.VMEM((NP,), jnp.float32),
        pltpu.VMEM((40, 128), jnp.int32),
        pltpu.VMEM((40, 128), jnp.int32),
        pltpu.VMEM((40, 128), jnp.float32),
        pltpu.VMEM((40, 128), jnp.float32),
        pltpu.VMEM((320,), jnp.float32),
        pltpu.VMEM((640,), jnp.float32),
    ],
)
def _sc_norm(srcR_hbm, dstR_hbm, ewR_hbm, normR_hbm, dinv2_hbm,
             deg_sh, dst_v, ew_v, dinv_v, srcn_v, dstn_v, ewn_v,
             norm_v, d2_v, zb):
    c = lax.axis_index("c")
    s = lax.axis_index("s")
    w = s * NCORE + c

    @pl.loop(0, 40)
    def _(k):
        zb[pl.ds(k * 16, 16)] = jnp.zeros((16,), jnp.float32)

    pltpu.sync_copy(zb, deg_sh.at[pl.ds(s * 640, 640)])
    plsc.subcore_barrier()

    pltpu.sync_copy(dstR_hbm.at[pl.ds(s * RPT, RPT)], dst_v)
    pltpu.sync_copy(ewR_hbm.at[pl.ds(s * RPT, RPT)], ew_v)

    @pl.loop(0, RPT)
    def _(k):
        pltpu.sync_copy(ew_v.at[k], deg_sh.at[dst_v.at[k]], add=True)

    plsc.subcore_barrier()

    pltpu.sync_copy(deg_sh, dinv_v)

    @plsc.parallel_loop(0, NP // 16, unroll=2)
    def _(k):
        sl = pl.ds(k * 16, 16)
        d = dinv_v[sl] + 1.0
        i = plsc.bitcast(d, jnp.int32)
        y = plsc.bitcast(jnp.full((16,), 0x5F3759DF, jnp.int32)
                         - lax.shift_right_logical(i, 1), jnp.float32)
        y = y * (1.5 - 0.5 * d * y * y)
        y = y * (1.5 - 0.5 * d * y * y)
        y = y * (1.5 - 0.5 * d * y * y)
        y = y * (1.5 - 0.5 * d * y * y)
        dinv_v[sl] = y

    @pl.loop(0, 20)
    def _(k):
        v = dinv_v[pl.ds(w * 320 + k * 16, 16)]
        d2_v[pl.ds(k * 16, 16)] = v * v

    pltpu.sync_copy(d2_v, dinv2_hbm.at[pl.ds(w * 320, 320)])

    pltpu.sync_copy(srcR_hbm.at[pl.ds(w * 40, 40)], srcn_v)
    pltpu.sync_copy(dstR_hbm.at[pl.ds(w * 40, 40)], dstn_v)
    pltpu.sync_copy(ewR_hbm.at[pl.ds(w * 40, 40)], ewn_v)

    @pl.loop(0, 40)
    def _(k):
        for q in range(8):
            sl = pl.ds(q * 16, 16)
            a = plsc.load_gather(dinv_v, [srcn_v[k, sl]])
            b = plsc.load_gather(dinv_v, [dstn_v[k, sl]])
            norm_v[k, sl] = a * ewn_v[k, sl] * b

    pltpu.sync_copy(norm_v, normR_hbm.at[pl.ds(w * 40, 40)])


def _make_agg(fh, edge_split):
    qg = fh // 16
    rpt = RPT // 2 if edge_split else RPT
    ng = rpt // 4

    @functools.partial(
        pl.kernel,
        out_type=jax.ShapeDtypeStruct((2 * N, fh), jnp.float32),
        mesh=_sc_mesh(),
        compiler_params=_SC_PARAMS,
        scratch_types=[
            pltpu.VMEM_SHARED((N, fh), jnp.float32),
            pltpu.VMEM((512,), jnp.int32),
            pltpu.VMEM((512,), jnp.int32),
            pltpu.VMEM((512,), jnp.float32),
            pltpu.VMEM((128, fh), jnp.float32),
            pltpu.VMEM((128, fh), jnp.float32),
            pltpu.SemaphoreType.DMA,
            pltpu.SemaphoreType.DMA,
            pltpu.SemaphoreType.DMA,
            pltpu.SemaphoreType.DMA,
        ],
    )
    def agg(h_hbm, srcF_hbm, dstF_hbm, normF_hbm, out_hbm,
            acc_sh, src_v, dst_v, norm_v, rows0, rows1, gs0, gs1, ss0, ss1):
        c = lax.axis_index("c")
        s = lax.axis_index("s")
        row0 = (s * NCORE + c) * rpt if edge_split else s * rpt

        cn = jnp.full((16,), (0 if edge_split else N) * c, jnp.int32)

        @pl.loop(0, 128)
        def _(r):
            for q in range(qg):
                rows0[r, pl.ds(q * 16, 16)] = jnp.zeros((16,), jnp.float32)

        for m, sz in enumerate((128, 128, 128, 128, 112)):
            pltpu.sync_copy(
                rows0.at[pl.ds(0, sz)],
                acc_sh.at[pl.ds(pl.multiple_of(s * 624 + m * 128, 8), sz)])

        @pl.when(s == NSUB - 1)
        def _():
            pltpu.sync_copy(rows0.at[pl.ds(0, 16)],
                            acc_sh.at[pl.ds(NSUB * 624, 16)])

        plsc.subcore_barrier()

        def stage(g):
            base = pl.multiple_of((row0 + g * 4) * 128, 8)
            pltpu.sync_copy(srcF_hbm.at[pl.ds(base, 512)], src_v)
            pltpu.sync_copy(dstF_hbm.at[pl.ds(base, 512)], dst_v)
            pltpu.sync_copy(normF_hbm.at[pl.ds(base, 512)], norm_v)

            @plsc.parallel_loop(0, 32, unroll=4)
            def _(k):
                sl = pl.ds(k * 16, 16)
                src_v[sl] = src_v[sl] + cn

        def scale(buf, jj):
            @plsc.parallel_loop(0, 128, unroll=8)
            def _(r):
                nv = plsc.load_gather(
                    norm_v, [jnp.full((16,), jj * 128 + r, jnp.int32)])
                for q in range(qg):
                    sl = pl.ds(q * 16, 16)
                    buf[r, sl] = buf[r, sl] * nv

        bufs = (rows0, rows1)
        sems = (gs0, gs1)
        ss = (ss0, ss1)

        def idx(jj):
            return pl.ds(jj * 128, 128)

        def wait_scatter(jj, p):
            pltpu.make_async_copy(
                bufs[p], acc_sh.at[dst_v.at[idx(jj)]], ss[p]).wait()

        stage(0)
        pltpu.async_copy(h_hbm.at[src_v.at[idx(0)]], rows0, gs0)

        @pl.loop(0, ng)
        def _(t):
            for jj in range(4):
                p = jj % 2
                buf, sem = bufs[p], sems[p]
                pltpu.make_async_copy(
                    h_hbm.at[src_v.at[idx(jj)]], buf, sem).wait()
                if jj >= 1:
                    wait_scatter(jj - 1, 1 - p)
                if jj < 3:
                    pltpu.async_copy(h_hbm.at[src_v.at[idx(jj + 1)]],
                                     bufs[1 - p], sems[1 - p])
                scale(buf, jj)
                pltpu.async_copy(buf, acc_sh.at[dst_v.at[idx(jj)]],
                                 ss[p], add=True)

            @pl.when(t < ng - 1)
            def _():
                wait_scatter(3, 1)
                stage(t + 1)
                pltpu.async_copy(h_hbm.at[src_v.at[idx(0)]], rows0, gs0)

        wait_scatter(3, 1)
        plsc.subcore_barrier()

        pltpu.sync_copy(
            acc_sh.at[pl.ds(pl.multiple_of(s * 624, 8), 624)],
            out_hbm.at[pl.ds(pl.multiple_of(c * N + s * 624, 8), 624)])

        @pl.when(s == NSUB - 1)
        def _():
            pltpu.sync_copy(acc_sh.at[pl.ds(NSUB * 624, 16)],
                            out_hbm.at[pl.ds(c * N + NSUB * 624, 16)])

    return agg


_agg128 = _make_agg(128, edge_split=False)
_agg3 = _make_agg(64, edge_split=True)


def _mm_split(x, W):
    n, d = x.shape
    f = W.shape[1]
    fh = f // 2
    bn = 2000
    nb = n // bn

    def body(x_ref, w_ref, o_ref):
        o_ref[...] = jnp.dot(x_ref[...], w_ref[...],
                             preferred_element_type=jnp.float32)

    return pl.pallas_call(
        body,
        grid=(nb, 2),
        in_specs=[pl.BlockSpec((bn, d), lambda i, j: (i, 0)),
                  pl.BlockSpec((d, fh), lambda i, j: (0, j))],
        out_specs=pl.BlockSpec((bn, fh), lambda i, j: (j * nb + i, 0)),
        out_shape=jax.ShapeDtypeStruct((2 * n, fh), jnp.float32),
    )(x, W)


def _post_bn_mm(agg, h, dinv2, b, g, be, W):
    f = b.shape[0]
    fh = f // 2
    fo = W.shape[1]
    foh = fo // 2
    bn = 2000
    nb = N // bn

    def body(a0, a1, h0, h1, d2, b_ref, g_ref, be_ref, w_ref, o_ref,
             z_scr, st_scr):
        i = pl.program_id(0)

        @pl.when(i < nb)
        def _():
            d2v = d2[...]
            z0 = a0[...] + h0[...] * d2v
            z1 = a1[...] + h1[...] * d2v
            z = jnp.concatenate([z0, z1], axis=1) + b_ref[...]
            z_scr[pl.ds(i * bn, bn), :] = z
            s0 = jnp.sum(z, axis=0, keepdims=True)
            s1 = jnp.sum(z * z, axis=0, keepdims=True)
            upd = jnp.concatenate(
                [s0, s1, jnp.zeros((6, f), jnp.float32)], axis=0)

            @pl.when(i == 0)
            def _():
                st_scr[...] = jnp.zeros_like(st_scr)

            st_scr[...] += upd

        @pl.when(i >= nb)
        def _():
            rb = lax.rem(i - nb, nb)
            mu = st_scr[0:1, :] * (1.0 / N)
            ex2 = st_scr[1:2, :] * (1.0 / N)
            var = ex2 - mu * mu
            istd = lax.rsqrt(var + 1e-5)
            z = z_scr[pl.ds(rb * bn, bn), :]
            y = jnp.maximum((z - mu) * istd * g_ref[...] + be_ref[...], 0.0)
            o_ref[...] = jnp.dot(y, w_ref[...],
                                 preferred_element_type=jnp.float32)

    p0 = lambda i: jnp.where(i < nb, i, 0)

    return pl.pallas_call(
        body,
        grid=(3 * nb,),
        in_specs=[
            pl.BlockSpec((bn, fh), lambda i, _p=p0: (_p(i), 0)),
            pl.BlockSpec((bn, fh), lambda i, _p=p0: (nb + _p(i), 0)),
            pl.BlockSpec((bn, fh), lambda i, _p=p0: (_p(i), 0)),
            pl.BlockSpec((bn, fh), lambda i, _p=p0: (nb + _p(i), 0)),
            pl.BlockSpec((bn, 1), lambda i, _p=p0: (_p(i), 0)),
            pl.BlockSpec((1, f), lambda i: (0, 0)),
            pl.BlockSpec((1, f), lambda i: (0, 0)),
            pl.BlockSpec((1, f), lambda i: (0, 0)),
            pl.BlockSpec((f, foh),
                         lambda i: (0, jnp.where(i < nb, 0, (i - nb) // nb))),
        ],
        out_specs=pl.BlockSpec(
            (bn, foh),
            lambda i: (jnp.where(i < nb, 0,
                                 lax.rem(i - nb, nb) + ((i - nb) // nb) * nb),
                       0)),
        out_shape=jax.ShapeDtypeStruct((2 * N, foh), jnp.float32),
        scratch_shapes=[pltpu.VMEM((N, f), jnp.float32),
                        pltpu.VMEM((8, f), jnp.float32)],
    )(agg, agg, h, h, dinv2, b.reshape(1, f), g.reshape(1, f),
      be.reshape(1, f), W)


def _post_bn_mm64(agg, h, dinv2, b, g, be, W):
    f = b.shape[0]
    fh = f // 2
    fo = W.shape[1]
    bn = 2000
    nb = N // bn

    def body(a0, a1, h0, h1, d2, b_ref, g_ref, be_ref, w_ref, o_ref,
             z_scr, st_scr):
        i = pl.program_id(0)

        @pl.when(i < nb)
        def _():
            d2v = d2[...]
            z0 = a0[...] + h0[...] * d2v
            z1 = a1[...] + h1[...] * d2v
            z = jnp.concatenate([z0, z1], axis=1) + b_ref[...]
            z_scr[pl.ds(i * bn, bn), :] = z
            s0 = jnp.sum(z, axis=0, keepdims=True)
            s1 = jnp.sum(z * z, axis=0, keepdims=True)
            upd = jnp.concatenate(
                [s0, s1, jnp.zeros((6, f), jnp.float32)], axis=0)

            @pl.when(i == 0)
            def _():
                st_scr[...] = jnp.zeros_like(st_scr)

            st_scr[...] += upd

        @pl.when(i >= nb)
        def _():
            rb = i - nb
            mu = st_scr[0:1, :] * (1.0 / N)
            ex2 = st_scr[1:2, :] * (1.0 / N)
            var = ex2 - mu * mu
            istd = lax.rsqrt(var + 1e-5)
            z = z_scr[pl.ds(rb * bn, bn), :]
            y = jnp.maximum((z - mu) * istd * g_ref[...] + be_ref[...], 0.0)
            o_ref[...] = jnp.dot(y, w_ref[...],
                                 preferred_element_type=jnp.float32)

    p0 = lambda i: jnp.where(i < nb, i, 0)

    return pl.pallas_call(
        body,
        grid=(2 * nb,),
        in_specs=[
            pl.BlockSpec((bn, fh), lambda i, _p=p0: (_p(i), 0)),
            pl.BlockSpec((bn, fh), lambda i, _p=p0: (nb + _p(i), 0)),
            pl.BlockSpec((bn, fh), lambda i, _p=p0: (_p(i), 0)),
            pl.BlockSpec((bn, fh), lambda i, _p=p0: (nb + _p(i), 0)),
            pl.BlockSpec((bn, 1), lambda i, _p=p0: (_p(i), 0)),
            pl.BlockSpec((1, f), lambda i: (0, 0)),
            pl.BlockSpec((1, f), lambda i: (0, 0)),
            pl.BlockSpec((1, f), lambda i: (0, 0)),
            pl.BlockSpec((f, fo), lambda i: (0, 0)),
        ],
        out_specs=pl.BlockSpec(
            (bn, fo), lambda i: (jnp.where(i < nb, 0, i - nb), 0)),
        out_shape=jax.ShapeDtypeStruct((N, fo), jnp.float32),
        scratch_shapes=[pltpu.VMEM((N, f), jnp.float32),
                        pltpu.VMEM((8, f), jnp.float32)],
    )(agg, agg, h, h, dinv2, b.reshape(1, f), g.reshape(1, f),
      be.reshape(1, f), W)


def _final(agg, h, dinv2, b):
    f = b.shape[0]
    bn = 2000
    nb = N // bn

    def body(a0, a1, h_ref, d2, b_ref, o_ref):
        z = a0[...] + a1[...] + h_ref[...] * d2[...] + b_ref[...]
        m = jnp.max(z, axis=1, keepdims=True)
        e = jnp.exp(z - m)
        lse = jnp.log(jnp.sum(e, axis=1, keepdims=True)) + m
        o_ref[...] = z - lse

    return pl.pallas_call(
        body,
        grid=(nb,),
        in_specs=[pl.BlockSpec((bn, f), lambda i: (i, 0)),
                  pl.BlockSpec((bn, f), lambda i: (nb + i, 0)),
                  pl.BlockSpec((bn, f), lambda i: (i, 0)),
                  pl.BlockSpec((bn, 1), lambda i: (i, 0)),
                  pl.BlockSpec((1, f), lambda i: (0, 0))],
        out_specs=pl.BlockSpec((bn, f), lambda i: (i, 0)),
        out_shape=jax.ShapeDtypeStruct((N, f), jnp.float32),
    )(agg, agg, h, dinv2, b.reshape(1, f))


def kernel(x, edge_index, edge_weight, W1, b1, g1, be1, W2, b2, g2, be2, W3, b3):
    src = edge_index[0]
    dst = edge_index[1]
    pad = EP - E
    srcR = jnp.concatenate(
        [src, jnp.zeros((pad,), jnp.int32)]).reshape(ER, 128)
    dstR = jnp.concatenate(
        [dst, jnp.zeros((pad,), jnp.int32)]).reshape(ER, 128)
    ewR = jnp.concatenate(
        [edge_weight, jnp.zeros((pad,), jnp.float32)]).reshape(ER, 128)

    normR, dinv2p = _sc_norm(srcR, dstR, ewR)
    dinv2 = dinv2p[:N].reshape(N, 1)
    srcF = srcR.reshape(EP)
    dstF = dstR.reshape(EP)
    normF = normR.reshape(EP)

    h1 = _mm_split(x, W1)
    a1 = _agg128(h1, srcF, dstF, normF)
    h2 = _post_bn_mm(a1, h1, dinv2, b1, g1, be1, W2)
    a2 = _agg128(h2, srcF, dstF, normF)
    h3 = _post_bn_mm64(a2, h2, dinv2, b2, g2, be2, W3)
    a3 = _agg3(h3, srcF, dstF, normF)
    return _final(a3, h3, dinv2, b3)


# speedup vs baseline: 1.0966x; 1.0966x over previous
import functools

import jax
import jax.numpy as jnp
from jax import lax
from jax.experimental import pallas as pl
from jax.experimental.pallas import tpu as pltpu
from jax.experimental.pallas import tpu_sc as plsc

N = 10000
NP = 10240
E = 160000
EP = 163840
ER = EP // 128
RPT = ER // 16
NSUB = 16
NCORE = 2


def _sc_mesh():
    return plsc.VectorSubcoreMesh(core_axis_name="c", subcore_axis_name="s")


_SC_PARAMS = pltpu.CompilerParams(needs_layout_passes=False,
                                  use_tc_tiling_on_sc=False)


@functools.partial(
    pl.kernel,
    out_type=(
        jax.ShapeDtypeStruct((ER, 128), jnp.float32),
        jax.ShapeDtypeStruct((NP,), jnp.float32),
    ),
    mesh=_sc_mesh(),
    compiler_params=_SC_PARAMS,
    scratch_types=[
        pltpu.VMEM_SHARED((NP,), jnp.float32),
        pltpu.VMEM((RPT, 128), jnp.int32),
        pltpu.VMEM((RPT, 128), jnp.float32),
        pltpu.VMEM((NP,), jnp.float32),
        pltpu.VMEM((40, 128), jnp.int32),
        pltpu.VMEM((40, 128), jnp.int32),
        pltpu.VMEM((40, 128), jnp.float32),
        pltpu.VMEM((40, 128), jnp.float32),
        pltpu.VMEM((320,), jnp.float32),
        pltpu.VMEM((640,), jnp.float32),
    ],
)
def _sc_norm(srcR_hbm, dstR_hbm, ewR_hbm, normR_hbm, dinv2_hbm,
             deg_sh, dst_v, ew_v, dinv_v, srcn_v, dstn_v, ewn_v,
             norm_v, d2_v, zb):
    c = lax.axis_index("c")
    s = lax.axis_index("s")
    w = s * NCORE + c

    @pl.loop(0, 40)
    def _(k):
        zb[pl.ds(k * 16, 16)] = jnp.zeros((16,), jnp.float32)

    pltpu.sync_copy(zb, deg_sh.at[pl.ds(s * 640, 640)])
    plsc.subcore_barrier()

    pltpu.sync_copy(dstR_hbm.at[pl.ds(s * RPT, RPT)], dst_v)
    pltpu.sync_copy(ewR_hbm.at[pl.ds(s * RPT, RPT)], ew_v)

    @pl.loop(0, RPT)
    def _(k):
        pltpu.sync_copy(ew_v.at[k], deg_sh.at[dst_v.at[k]], add=True)

    plsc.subcore_barrier()

    pltpu.sync_copy(deg_sh, dinv_v)

    @plsc.parallel_loop(0, NP // 16, unroll=2)
    def _(k):
        sl = pl.ds(k * 16, 16)
        d = dinv_v[sl] + 1.0
        i = plsc.bitcast(d, jnp.int32)
        y = plsc.bitcast(jnp.full((16,), 0x5F3759DF, jnp.int32)
                         - lax.shift_right_logical(i, 1), jnp.float32)
        y = y * (1.5 - 0.5 * d * y * y)
        y = y * (1.5 - 0.5 * d * y * y)
        y = y * (1.5 - 0.5 * d * y * y)
        y = y * (1.5 - 0.5 * d * y * y)
        dinv_v[sl] = y

    @pl.loop(0, 20)
    def _(k):
        v = dinv_v[pl.ds(w * 320 + k * 16, 16)]
        d2_v[pl.ds(k * 16, 16)] = v * v

    pltpu.sync_copy(d2_v, dinv2_hbm.at[pl.ds(w * 320, 320)])

    pltpu.sync_copy(srcR_hbm.at[pl.ds(w * 40, 40)], srcn_v)
    pltpu.sync_copy(dstR_hbm.at[pl.ds(w * 40, 40)], dstn_v)
    pltpu.sync_copy(ewR_hbm.at[pl.ds(w * 40, 40)], ewn_v)

    @pl.loop(0, 40)
    def _(k):
        for q in range(8):
            sl = pl.ds(q * 16, 16)
            a = plsc.load_gather(dinv_v, [srcn_v[k, sl]])
            b = plsc.load_gather(dinv_v, [dstn_v[k, sl]])
            norm_v[k, sl] = a * ewn_v[k, sl] * b

    pltpu.sync_copy(norm_v, normR_hbm.at[pl.ds(w * 40, 40)])


def _make_agg(fh, edge_split):
    qg = fh // 16
    rpt = RPT // 2 if edge_split else RPT
    ng = rpt // 4

    @functools.partial(
        pl.kernel,
        out_type=jax.ShapeDtypeStruct((2 * N, fh), jnp.float32),
        mesh=_sc_mesh(),
        compiler_params=_SC_PARAMS,
        scratch_types=[
            pltpu.VMEM_SHARED((N, fh), jnp.float32),
            pltpu.VMEM((512,), jnp.int32),
            pltpu.VMEM((512,), jnp.int32),
            pltpu.VMEM((512,), jnp.float32),
            pltpu.VMEM((128, fh), jnp.float32),
            pltpu.VMEM((128, fh), jnp.float32),
            pltpu.SemaphoreType.DMA,
            pltpu.SemaphoreType.DMA,
            pltpu.SemaphoreType.DMA,
            pltpu.SemaphoreType.DMA,
            pltpu.SemaphoreType.DMA,
        ],
    )
    def agg(h_hbm, srcF_hbm, dstF_hbm, normF_hbm, out_hbm,
            acc_sh, src_v, dst_v, norm_v, rows0, rows1,
            gs0, gs1, ss0, ss1, isem):
        c = lax.axis_index("c")
        s = lax.axis_index("s")
        row0 = (s * NCORE + c) * rpt if edge_split else s * rpt

        cn = jnp.full((16,), (0 if edge_split else N) * c, jnp.int32)

        @pl.loop(0, 128)
        def _(r):
            for q in range(qg):
                rows0[r, pl.ds(q * 16, 16)] = jnp.zeros((16,), jnp.float32)

        for m, sz in enumerate((128, 128, 128, 128, 112)):
            pltpu.sync_copy(
                rows0.at[pl.ds(0, sz)],
                acc_sh.at[pl.ds(pl.multiple_of(s * 624 + m * 128, 8), sz)])

        @pl.when(s == NSUB - 1)
        def _():
            pltpu.sync_copy(rows0.at[pl.ds(0, 16)],
                            acc_sh.at[pl.ds(NSUB * 624, 16)])

        plsc.subcore_barrier()

        def stage_src(g):
            base = pl.multiple_of((row0 + g * 4) * 128, 8)
            pltpu.sync_copy(srcF_hbm.at[pl.ds(base, 512)], src_v)

            @plsc.parallel_loop(0, 32, unroll=4)
            def _(k):
                sl = pl.ds(k * 16, 16)
                src_v[sl] = src_v[sl] + cn

        def stage_dn(g, start):
            base = pl.multiple_of((row0 + g * 4) * 128, 8)
            cp = (pltpu.async_copy if start else pltpu.make_async_copy)
            d1 = cp(dstF_hbm.at[pl.ds(base, 512)], dst_v, isem)
            d2 = cp(normF_hbm.at[pl.ds(base, 512)], norm_v, isem)
            if not start:
                d1.wait()
                d2.wait()

        def scale(buf, jj):
            @plsc.parallel_loop(0, 128, unroll=8)
            def _(r):
                nv = plsc.load_gather(
                    norm_v, [jnp.full((16,), jj * 128 + r, jnp.int32)])
                for q in range(qg):
                    sl = pl.ds(q * 16, 16)
                    buf[r, sl] = buf[r, sl] * nv

        bufs = (rows0, rows1)
        sems = (gs0, gs1)
        ss = (ss0, ss1)

        def idx(jj):
            return pl.ds(jj * 128, 128)

        def wait_scatter(jj, p):
            pltpu.make_async_copy(
                bufs[p], acc_sh.at[dst_v.at[idx(jj)]], ss[p]).wait()

        stage_src(0)
        pltpu.async_copy(h_hbm.at[src_v.at[idx(0)]], rows0, gs0)
        stage_dn(0, start=True)

        @pl.loop(0, ng)
        def _(t):
            for jj in range(4):
                p = jj % 2
                buf, sem = bufs[p], sems[p]
                pltpu.make_async_copy(
                    h_hbm.at[src_v.at[idx(jj)]], buf, sem).wait()
                if jj == 0:
                    stage_dn(t, start=False)
                if jj >= 1:
                    wait_scatter(jj - 1, 1 - p)
                if jj < 3:
                    pltpu.async_copy(h_hbm.at[src_v.at[idx(jj + 1)]],
                                     bufs[1 - p], sems[1 - p])
                scale(buf, jj)
                pltpu.async_copy(buf, acc_sh.at[dst_v.at[idx(jj)]],
                                 ss[p], add=True)

            @pl.when(t < ng - 1)
            def _():
                wait_scatter(3, 1)
                stage_src(t + 1)
                pltpu.async_copy(h_hbm.at[src_v.at[idx(0)]], rows0, gs0)
                stage_dn(t + 1, start=True)

        wait_scatter(3, 1)
        plsc.subcore_barrier()

        pltpu.sync_copy(
            acc_sh.at[pl.ds(pl.multiple_of(s * 624, 8), 624)],
            out_hbm.at[pl.ds(pl.multiple_of(c * N + s * 624, 8), 624)])

        @pl.when(s == NSUB - 1)
        def _():
            pltpu.sync_copy(acc_sh.at[pl.ds(NSUB * 624, 16)],
                            out_hbm.at[pl.ds(c * N + NSUB * 624, 16)])

    return agg


_agg128 = _make_agg(128, edge_split=False)
_agg3 = _make_agg(64, edge_split=True)


def _mm_split(x, W):
    n, d = x.shape
    f = W.shape[1]
    fh = f // 2
    bn = 2000
    nb = n // bn

    def body(x_ref, w_ref, o_ref):
        o_ref[...] = jnp.dot(x_ref[...], w_ref[...],
                             preferred_element_type=jnp.float32)

    return pl.pallas_call(
        body,
        grid=(nb, 2),
        in_specs=[pl.BlockSpec((bn, d), lambda i, j: (i, 0)),
                  pl.BlockSpec((d, fh), lambda i, j: (0, j))],
        out_specs=pl.BlockSpec((bn, fh), lambda i, j: (j * nb + i, 0)),
        out_shape=jax.ShapeDtypeStruct((2 * n, fh), jnp.float32),
    )(x, W)


def _post(agg, h, dinv2, b):
    f = b.shape[0]
    fh = f // 2
    bn = 2000
    nb = N // bn

    def body(a0, a1, h0, h1, d2, b_ref, z_ref, st_ref):
        d2v = d2[...]
        z0 = a0[...] + h0[...] * d2v
        z1 = a1[...] + h1[...] * d2v
        z = jnp.concatenate([z0, z1], axis=1) + b_ref[...]
        z_ref[...] = z
        s0 = jnp.sum(z, axis=0, keepdims=True)
        s1 = jnp.sum(z * z, axis=0, keepdims=True)
        upd = jnp.concatenate([s0, s1, jnp.zeros((6, f), jnp.float32)], axis=0)

        @pl.when(pl.program_id(0) == 0)
        def _():
            st_ref[...] = jnp.zeros_like(st_ref)

        st_ref[...] += upd

    return pl.pallas_call(
        body,
        grid=(nb,),
        in_specs=[pl.BlockSpec((bn, fh), lambda i: (i, 0)),
                  pl.BlockSpec((bn, fh), lambda i: (nb + i, 0)),
                  pl.BlockSpec((bn, fh), lambda i: (i, 0)),
                  pl.BlockSpec((bn, fh), lambda i: (nb + i, 0)),
                  pl.BlockSpec((bn, 1), lambda i: (i, 0)),
                  pl.BlockSpec((1, f), lambda i: (0, 0))],
        out_specs=[pl.BlockSpec((bn, f), lambda i: (i, 0)),
                   pl.BlockSpec((8, f), lambda i: (0, 0))],
        out_shape=[jax.ShapeDtypeStruct((N, f), jnp.float32),
                   jax.ShapeDtypeStruct((8, f), jnp.float32)],
    )(agg, agg, h, h, dinv2, b.reshape(1, f))


def _bn_relu_mm(z, st, g, be, W, split):
    n, f = z.shape
    fo = W.shape[1]
    fh = fo // 2 if split else fo
    bn = 2000
    nb = n // bn

    def body(z_ref, st_ref, g_ref, be_ref, w_ref, o_ref):
        mu = st_ref[0:1, :] * (1.0 / n)
        ex2 = st_ref[1:2, :] * (1.0 / n)
        var = ex2 - mu * mu
        istd = lax.rsqrt(var + 1e-5)
        y = jnp.maximum((z_ref[...] - mu) * istd * g_ref[...] + be_ref[...],
                        0.0)
        o_ref[...] = jnp.dot(y, w_ref[...], preferred_element_type=jnp.float32)

    if split:
        grid = (nb, 2)
        w_spec = pl.BlockSpec((f, fh), lambda i, j: (0, j))
        o_spec = pl.BlockSpec((bn, fh), lambda i, j: (j * nb + i, 0))
        o_shape = jax.ShapeDtypeStruct((2 * n, fh), jnp.float32)
        bcast = lambda m: pl.BlockSpec(m, lambda i, j: (0, 0))
        z_spec = pl.BlockSpec((bn, f), lambda i, j: (i, 0))
    else:
        grid = (nb,)
        w_spec = pl.BlockSpec((f, fh), lambda i: (0, 0))
        o_spec = pl.BlockSpec((bn, fh), lambda i: (i, 0))
        o_shape = jax.ShapeDtypeStruct((n, fh), jnp.float32)
        bcast = lambda m: pl.BlockSpec(m, lambda i: (0, 0))
        z_spec = pl.BlockSpec((bn, f), lambda i: (i, 0))

    return pl.pallas_call(
        body,
        grid=grid,
        in_specs=[z_spec, bcast((8, f)), bcast((1, f)), bcast((1, f)), w_spec],
        out_specs=o_spec,
        out_shape=o_shape,
    )(z, st, g.reshape(1, f), be.reshape(1, f), W)


def _final(agg, h, dinv2, b):
    f = b.shape[0]
    bn = 2000
    nb = N // bn

    def body(a0, a1, h_ref, d2, b_ref, o_ref):
        z = a0[...] + a1[...] + h_ref[...] * d2[...] + b_ref[...]
        m = jnp.max(z, axis=1, keepdims=True)
        e = jnp.exp(z - m)
        lse = jnp.log(jnp.sum(e, axis=1, keepdims=True)) + m
        o_ref[...] = z - lse

    return pl.pallas_call(
        body,
        grid=(nb,),
        in_specs=[pl.BlockSpec((bn, f), lambda i: (i, 0)),
                  pl.BlockSpec((bn, f), lambda i: (nb + i, 0)),
                  pl.BlockSpec((bn, f), lambda i: (i, 0)),
                  pl.BlockSpec((bn, 1), lambda i: (i, 0)),
                  pl.BlockSpec((1, f), lambda i: (0, 0))],
        out_specs=pl.BlockSpec((bn, f), lambda i: (i, 0)),
        out_shape=jax.ShapeDtypeStruct((N, f), jnp.float32),
    )(agg, agg, h, dinv2, b.reshape(1, f))


def kernel(x, edge_index, edge_weight, W1, b1, g1, be1, W2, b2, g2, be2, W3, b3):
    src = edge_index[0]
    dst = edge_index[1]
    pad = EP - E
    srcR = jnp.concatenate(
        [src, jnp.zeros((pad,), jnp.int32)]).reshape(ER, 128)
    dstR = jnp.concatenate(
        [dst, jnp.zeros((pad,), jnp.int32)]).reshape(ER, 128)
    ewR = jnp.concatenate(
        [edge_weight, jnp.zeros((pad,), jnp.float32)]).reshape(ER, 128)

    normR, dinv2p = _sc_norm(srcR, dstR, ewR)
    dinv2 = dinv2p[:N].reshape(N, 1)
    srcF = srcR.reshape(EP)
    dstF = dstR.reshape(EP)
    normF = normR.reshape(EP)

    h1 = _mm_split(x, W1)
    a1 = _agg128(h1, srcF, dstF, normF)
    z1, st1 = _post(a1, h1, dinv2, b1)
    h2 = _bn_relu_mm(z1, st1, g1, be1, W2, split=True)
    a2 = _agg128(h2, srcF, dstF, normF)
    z2, st2 = _post(a2, h2, dinv2, b2)
    h3 = _bn_relu_mm(z2, st2, g2, be2, W3, split=False)
    a3 = _agg3(h3, srcF, dstF, normF)
    return _final(a3, h3, dinv2, b3)


# speedup vs baseline: 1.1218x; 1.0230x over previous
import functools

import jax
import jax.numpy as jnp
from jax import lax
from jax.experimental import pallas as pl
from jax.experimental.pallas import tpu as pltpu
from jax.experimental.pallas import tpu_sc as plsc

N = 10000
NP = 10240
E = 160000
EP = 163840
ER = EP // 128
RPT = ER // 16
NSUB = 16
NCORE = 2


def _sc_mesh():
    return plsc.VectorSubcoreMesh(core_axis_name="c", subcore_axis_name="s")


_SC_PARAMS = pltpu.CompilerParams(needs_layout_passes=False,
                                  use_tc_tiling_on_sc=False)


@functools.partial(
    pl.kernel,
    out_type=(
        jax.ShapeDtypeStruct((ER, 128), jnp.float32),
        jax.ShapeDtypeStruct((NP,), jnp.float32),
    ),
    mesh=_sc_mesh(),
    compiler_params=_SC_PARAMS,
    scratch_types=[
        pltpu.VMEM_SHARED((NP,), jnp.float32),
        pltpu.VMEM((RPT, 128), jnp.int32),
        pltpu.VMEM((RPT, 128), jnp.float32),
        pltpu.VMEM((NP,), jnp.float32),
        pltpu.VMEM((40, 128), jnp.int32),
        pltpu.VMEM((40, 128), jnp.int32),
        pltpu.VMEM((40, 128), jnp.float32),
        pltpu.VMEM((40, 128), jnp.float32),
        pltpu.VMEM((320,), jnp.float32),
        pltpu.VMEM((640,), jnp.float32),
    ],
)
def _sc_norm(srcR_hbm, dstR_hbm, ewR_hbm, normR_hbm, dinv2_hbm,
             deg_sh, dst_v, ew_v, dinv_v, srcn_v, dstn_v, ewn_v,
             norm_v, d2_v, zb):
    c = lax.axis_index("c")
    s = lax.axis_index("s")
    w = s * NCORE + c

    @pl.loop(0, 40)
    def _(k):
        zb[pl.ds(k * 16, 16)] = jnp.zeros((16,), jnp.float32)

    pltpu.sync_copy(zb, deg_sh.at[pl.ds(s * 640, 640)])
    plsc.subcore_barrier()

    pltpu.sync_copy(dstR_hbm.at[pl.ds(s * RPT, RPT)], dst_v)
    pltpu.sync_copy(ewR_hbm.at[pl.ds(s * RPT, RPT)], ew_v)

    @pl.loop(0, RPT)
    def _(k):
        pltpu.sync_copy(ew_v.at[k], deg_sh.at[dst_v.at[k]], add=True)

    plsc.subcore_barrier()

    pltpu.sync_copy(deg_sh, dinv_v)

    @plsc.parallel_loop(0, NP // 16, unroll=2)
    def _(k):
        sl = pl.ds(k * 16, 16)
        d = dinv_v[sl] + 1.0
        i = plsc.bitcast(d, jnp.int32)
        y = plsc.bitcast(jnp.full((16,), 0x5F3759DF, jnp.int32)
                         - lax.shift_right_logical(i, 1), jnp.float32)
        y = y * (1.5 - 0.5 * d * y * y)
        y = y * (1.5 - 0.5 * d * y * y)
        y = y * (1.5 - 0.5 * d * y * y)
        y = y * (1.5 - 0.5 * d * y * y)
        dinv_v[sl] = y

    @pl.loop(0, 20)
    def _(k):
        v = dinv_v[pl.ds(w * 320 + k * 16, 16)]
        d2_v[pl.ds(k * 16, 16)] = v * v

    pltpu.sync_copy(d2_v, dinv2_hbm.at[pl.ds(w * 320, 320)])

    pltpu.sync_copy(srcR_hbm.at[pl.ds(w * 40, 40)], srcn_v)
    pltpu.sync_copy(dstR_hbm.at[pl.ds(w * 40, 40)], dstn_v)
    pltpu.sync_copy(ewR_hbm.at[pl.ds(w * 40, 40)], ewn_v)

    @pl.loop(0, 40)
    def _(k):
        for q in range(8):
            sl = pl.ds(q * 16, 16)
            a = plsc.load_gather(dinv_v, [srcn_v[k, sl]])
            b = plsc.load_gather(dinv_v, [dstn_v[k, sl]])
            norm_v[k, sl] = a * ewn_v[k, sl] * b

    pltpu.sync_copy(norm_v, normR_hbm.at[pl.ds(w * 40, 40)])


def _make_agg(fh, edge_split):
    qg = fh // 16
    rpt = RPT // 2 if edge_split else RPT
    ng = rpt // 4

    @functools.partial(
        pl.kernel,
        out_type=jax.ShapeDtypeStruct((2 * N, fh), jnp.float32),
        mesh=_sc_mesh(),
        compiler_params=_SC_PARAMS,
        scratch_types=[
            pltpu.VMEM_SHARED((N, fh), jnp.float32),
            pltpu.VMEM((512,), jnp.int32),
            pltpu.VMEM((512,), jnp.int32),
            pltpu.VMEM((512,), jnp.float32),
            pltpu.VMEM((128, fh), jnp.float32),
            pltpu.VMEM((128, fh), jnp.float32),
            pltpu.SemaphoreType.DMA,
            pltpu.SemaphoreType.DMA,
            pltpu.SemaphoreType.DMA,
            pltpu.SemaphoreType.DMA,
            pltpu.SemaphoreType.DMA,
            pltpu.SemaphoreType.DMA,
        ],
    )
    def agg(h_hbm, srcF_hbm, dstF_hbm, normF_hbm, out_hbm,
            acc_sh, src_v, dst_v, norm_v, rows0, rows1,
            gs0, gs1, ss0, ss1, isem, isem2):
        c = lax.axis_index("c")
        s = lax.axis_index("s")
        row0 = (s * NCORE + c) * rpt if edge_split else s * rpt

        cn = jnp.full((16,), (0 if edge_split else N) * c, jnp.int32)

        @pl.loop(0, 128)
        def _(r):
            for q in range(qg):
                rows0[r, pl.ds(q * 16, 16)] = jnp.zeros((16,), jnp.float32)

        for m, sz in enumerate((128, 128, 128, 128, 112)):
            pltpu.sync_copy(
                rows0.at[pl.ds(0, sz)],
                acc_sh.at[pl.ds(pl.multiple_of(s * 624 + m * 128, 8), sz)])

        @pl.when(s == NSUB - 1)
        def _():
            pltpu.sync_copy(rows0.at[pl.ds(0, 16)],
                            acc_sh.at[pl.ds(NSUB * 624, 16)])

        plsc.subcore_barrier()

        def stage_src(g, start):
            base = pl.multiple_of((row0 + g * 4) * 128, 8)
            cp = (pltpu.async_copy if start else pltpu.make_async_copy)
            d = cp(srcF_hbm.at[pl.ds(base, 512)], src_v, isem2)
            if start:
                return
            d.wait()

            @plsc.parallel_loop(0, 32, unroll=4)
            def _(k):
                sl = pl.ds(k * 16, 16)
                src_v[sl] = src_v[sl] + cn

        def stage_dn(g, start):
            base = pl.multiple_of((row0 + g * 4) * 128, 8)
            cp = (pltpu.async_copy if start else pltpu.make_async_copy)
            d1 = cp(dstF_hbm.at[pl.ds(base, 512)], dst_v, isem)
            d2 = cp(normF_hbm.at[pl.ds(base, 512)], norm_v, isem)
            if not start:
                d1.wait()
                d2.wait()

        def scale(buf, jj):
            @plsc.parallel_loop(0, 128, unroll=8)
            def _(r):
                nv = plsc.load_gather(
                    norm_v, [jnp.full((16,), jj * 128 + r, jnp.int32)])
                for q in range(qg):
                    sl = pl.ds(q * 16, 16)
                    buf[r, sl] = buf[r, sl] * nv

        bufs = (rows0, rows1)
        sems = (gs0, gs1)
        ss = (ss0, ss1)

        def idx(jj):
            return pl.ds(jj * 128, 128)

        def wait_scatter(jj, p):
            pltpu.make_async_copy(
                bufs[p], acc_sh.at[dst_v.at[idx(jj)]], ss[p]).wait()

        stage_src(0, start=True)
        stage_src(0, start=False)
        pltpu.async_copy(h_hbm.at[src_v.at[idx(0)]], rows0, gs0)
        stage_dn(0, start=True)

        @pl.loop(0, ng)
        def _(t):
            for jj in range(4):
                p = jj % 2
                buf, sem = bufs[p], sems[p]
                pltpu.make_async_copy(
                    h_hbm.at[src_v.at[idx(jj)]], buf, sem).wait()
                if jj == 0:
                    stage_dn(t, start=False)
                if jj >= 1:
                    wait_scatter(jj - 1, 1 - p)
                if jj < 3:
                    pltpu.async_copy(h_hbm.at[src_v.at[idx(jj + 1)]],
                                     bufs[1 - p], sems[1 - p])
                if jj == 3:
                    @pl.when(t < ng - 1)
                    def _():
                        stage_src(t + 1, start=True)
                scale(buf, jj)
                pltpu.async_copy(buf, acc_sh.at[dst_v.at[idx(jj)]],
                                 ss[p], add=True)

            @pl.when(t < ng - 1)
            def _():
                wait_scatter(3, 1)
                stage_src(t + 1, start=False)
                pltpu.async_copy(h_hbm.at[src_v.at[idx(0)]], rows0, gs0)
                stage_dn(t + 1, start=True)

        wait_scatter(3, 1)
        plsc.subcore_barrier()

        pltpu.sync_copy(
            acc_sh.at[pl.ds(pl.multiple_of(s * 624, 8), 624)],
            out_hbm.at[pl.ds(pl.multiple_of(c * N + s * 624, 8), 624)])

        @pl.when(s == NSUB - 1)
        def _():
            pltpu.sync_copy(acc_sh.at[pl.ds(NSUB * 624, 16)],
                            out_hbm.at[pl.ds(c * N + NSUB * 624, 16)])

    return agg


_agg128 = _make_agg(128, edge_split=False)
_agg3 = _make_agg(64, edge_split=True)


def _mm_split(x, W):
    n, d = x.shape
    f = W.shape[1]
    fh = f // 2
    bn = 2000
    nb = n // bn

    def body(x_ref, w_ref, o_ref):
        o_ref[...] = jnp.dot(x_ref[...], w_ref[...],
                             preferred_element_type=jnp.float32)

    return pl.pallas_call(
        body,
        grid=(nb, 2),
        in_specs=[pl.BlockSpec((bn, d), lambda i, j: (i, 0)),
                  pl.BlockSpec((d, fh), lambda i, j: (0, j))],
        out_specs=pl.BlockSpec((bn, fh), lambda i, j: (j * nb + i, 0)),
        out_shape=jax.ShapeDtypeStruct((2 * n, fh), jnp.float32),
    )(x, W)


def _post(agg, h, dinv2, b):
    f = b.shape[0]
    fh = f // 2
    bn = 2000
    nb = N // bn

    def body(a0, a1, h0, h1, d2, b_ref, z_ref, st_ref):
        d2v = d2[...]
        z0 = a0[...] + h0[...] * d2v
        z1 = a1[...] + h1[...] * d2v
        z = jnp.concatenate([z0, z1], axis=1) + b_ref[...]
        z_ref[...] = z
        s0 = jnp.sum(z, axis=0, keepdims=True)
        s1 = jnp.sum(z * z, axis=0, keepdims=True)
        upd = jnp.concatenate([s0, s1, jnp.zeros((6, f), jnp.float32)], axis=0)

        @pl.when(pl.program_id(0) == 0)
        def _():
            st_ref[...] = jnp.zeros_like(st_ref)

        st_ref[...] += upd

    return pl.pallas_call(
        body,
        grid=(nb,),
        in_specs=[pl.BlockSpec((bn, fh), lambda i: (i, 0)),
                  pl.BlockSpec((bn, fh), lambda i: (nb + i, 0)),
                  pl.BlockSpec((bn, fh), lambda i: (i, 0)),
                  pl.BlockSpec((bn, fh), lambda i: (nb + i, 0)),
                  pl.BlockSpec((bn, 1), lambda i: (i, 0)),
                  pl.BlockSpec((1, f), lambda i: (0, 0))],
        out_specs=[pl.BlockSpec((bn, f), lambda i: (i, 0)),
                   pl.BlockSpec((8, f), lambda i: (0, 0))],
        out_shape=[jax.ShapeDtypeStruct((N, f), jnp.float32),
                   jax.ShapeDtypeStruct((8, f), jnp.float32)],
    )(agg, agg, h, h, dinv2, b.reshape(1, f))


def _bn_relu_mm(z, st, g, be, W, split):
    n, f = z.shape
    fo = W.shape[1]
    fh = fo // 2 if split else fo
    bn = 2000
    nb = n // bn

    def body(z_ref, st_ref, g_ref, be_ref, w_ref, o_ref):
        mu = st_ref[0:1, :] * (1.0 / n)
        ex2 = st_ref[1:2, :] * (1.0 / n)
        var = ex2 - mu * mu
        istd = lax.rsqrt(var + 1e-5)
        y = jnp.maximum((z_ref[...] - mu) * istd * g_ref[...] + be_ref[...],
                        0.0)
        o_ref[...] = jnp.dot(y, w_ref[...], preferred_element_type=jnp.float32)

    if split:
        grid = (nb, 2)
        w_spec = pl.BlockSpec((f, fh), lambda i, j: (0, j))
        o_spec = pl.BlockSpec((bn, fh), lambda i, j: (j * nb + i, 0))
        o_shape = jax.ShapeDtypeStruct((2 * n, fh), jnp.float32)
        bcast = lambda m: pl.BlockSpec(m, lambda i, j: (0, 0))
        z_spec = pl.BlockSpec((bn, f), lambda i, j: (i, 0))
    else:
        grid = (nb,)
        w_spec = pl.BlockSpec((f, fh), lambda i: (0, 0))
        o_spec = pl.BlockSpec((bn, fh), lambda i: (i, 0))
        o_shape = jax.ShapeDtypeStruct((n, fh), jnp.float32)
        bcast = lambda m: pl.BlockSpec(m, lambda i: (0, 0))
        z_spec = pl.BlockSpec((bn, f), lambda i: (i, 0))

    return pl.pallas_call(
        body,
        grid=grid,
        in_specs=[z_spec, bcast((8, f)), bcast((1, f)), bcast((1, f)), w_spec],
        out_specs=o_spec,
        out_shape=o_shape,
    )(z, st, g.reshape(1, f), be.reshape(1, f), W)


def _final(agg, h, dinv2, b):
    f = b.shape[0]
    bn = 2000
    nb = N // bn

    def body(a0, a1, h_ref, d2, b_ref, o_ref):
        z = a0[...] + a1[...] + h_ref[...] * d2[...] + b_ref[...]
        m = jnp.max(z, axis=1, keepdims=True)
        e = jnp.exp(z - m)
        lse = jnp.log(jnp.sum(e, axis=1, keepdims=True)) + m
        o_ref[...] = z - lse

    return pl.pallas_call(
        body,
        grid=(nb,),
        in_specs=[pl.BlockSpec((bn, f), lambda i: (i, 0)),
                  pl.BlockSpec((bn, f), lambda i: (nb + i, 0)),
                  pl.BlockSpec((bn, f), lambda i: (i, 0)),
                  pl.BlockSpec((bn, 1), lambda i: (i, 0)),
                  pl.BlockSpec((1, f), lambda i: (0, 0))],
        out_specs=pl.BlockSpec((bn, f), lambda i: (i, 0)),
        out_shape=jax.ShapeDtypeStruct((N, f), jnp.float32),
    )(agg, agg, h, dinv2, b.reshape(1, f))


def kernel(x, edge_index, edge_weight, W1, b1, g1, be1, W2, b2, g2, be2, W3, b3):
    src = edge_index[0]
    dst = edge_index[1]
    pad = EP - E
    srcR = jnp.concatenate(
        [src, jnp.zeros((pad,), jnp.int32)]).reshape(ER, 128)
    dstR = jnp.concatenate(
        [dst, jnp.zeros((pad,), jnp.int32)]).reshape(ER, 128)
    ewR = jnp.concatenate(
        [edge_weight, jnp.zeros((pad,), jnp.float32)]).reshape(ER, 128)

    normR, dinv2p = _sc_norm(srcR, dstR, ewR)
    dinv2 = dinv2p[:N].reshape(N, 1)
    srcF = srcR.reshape(EP)
    dstF = dstR.reshape(EP)
    normF = normR.reshape(EP)

    h1 = _mm_split(x, W1)
    a1 = _agg128(h1, srcF, dstF, normF)
    z1, st1 = _post(a1, h1, dinv2, b1)
    h2 = _bn_relu_mm(z1, st1, g1, be1, W2, split=True)
    a2 = _agg128(h2, srcF, dstF, normF)
    z2, st2 = _post(a2, h2, dinv2, b2)
    h3 = _bn_relu_mm(z2, st2, g2, be2, W3, split=False)
    a3 = _agg3(h3, srcF, dstF, normF)
    return _final(a3, h3, dinv2, b3)


# speedup vs baseline: 1.1490x; 1.0243x over previous
import functools

import jax
import jax.numpy as jnp
from jax import lax
from jax.experimental import pallas as pl
from jax.experimental.pallas import tpu as pltpu
from jax.experimental.pallas import tpu_sc as plsc

N = 10000
NP = 10240
E = 160000
EP = 163840
ER = EP // 128
RPT = ER // 16
NSUB = 16
NCORE = 2


def _sc_mesh():
    return plsc.VectorSubcoreMesh(core_axis_name="c", subcore_axis_name="s")


_SC_PARAMS = pltpu.CompilerParams(needs_layout_passes=False,
                                  use_tc_tiling_on_sc=False)


@functools.partial(
    pl.kernel,
    out_type=(
        jax.ShapeDtypeStruct((ER, 128), jnp.float32),
        jax.ShapeDtypeStruct((NP,), jnp.float32),
    ),
    mesh=_sc_mesh(),
    compiler_params=_SC_PARAMS,
    scratch_types=[
        pltpu.VMEM_SHARED((NP,), jnp.float32),
        pltpu.VMEM((RPT, 128), jnp.int32),
        pltpu.VMEM((RPT, 128), jnp.float32),
        pltpu.VMEM((NP,), jnp.float32),
        pltpu.VMEM((40, 128), jnp.int32),
        pltpu.VMEM((40, 128), jnp.int32),
        pltpu.VMEM((40, 128), jnp.float32),
        pltpu.VMEM((40, 128), jnp.float32),
        pltpu.VMEM((320,), jnp.float32),
        pltpu.VMEM((640,), jnp.float32),
    ],
)
def _sc_norm(srcR_hbm, dstR_hbm, ewR_hbm, normR_hbm, dinv2_hbm,
             deg_sh, dst_v, ew_v, dinv_v, srcn_v, dstn_v, ewn_v,
             norm_v, d2_v, zb):
    c = lax.axis_index("c")
    s = lax.axis_index("s")
    w = s * NCORE + c

    @pl.loop(0, 40)
    def _(k):
        zb[pl.ds(k * 16, 16)] = jnp.zeros((16,), jnp.float32)

    pltpu.sync_copy(zb, deg_sh.at[pl.ds(s * 640, 640)])
    plsc.subcore_barrier()

    pltpu.sync_copy(dstR_hbm.at[pl.ds(s * RPT, RPT)], dst_v)
    pltpu.sync_copy(ewR_hbm.at[pl.ds(s * RPT, RPT)], ew_v)

    @pl.loop(0, RPT)
    def _(k):
        pltpu.sync_copy(ew_v.at[k], deg_sh.at[dst_v.at[k]], add=True)

    plsc.subcore_barrier()

    pltpu.sync_copy(deg_sh, dinv_v)

    @plsc.parallel_loop(0, NP // 16, unroll=2)
    def _(k):
        sl = pl.ds(k * 16, 16)
        d = dinv_v[sl] + 1.0
        i = plsc.bitcast(d, jnp.int32)
        y = plsc.bitcast(jnp.full((16,), 0x5F3759DF, jnp.int32)
                         - lax.shift_right_logical(i, 1), jnp.float32)
        y = y * (1.5 - 0.5 * d * y * y)
        y = y * (1.5 - 0.5 * d * y * y)
        y = y * (1.5 - 0.5 * d * y * y)
        y = y * (1.5 - 0.5 * d * y * y)
        dinv_v[sl] = y

    @pl.loop(0, 20)
    def _(k):
        v = dinv_v[pl.ds(w * 320 + k * 16, 16)]
        d2_v[pl.ds(k * 16, 16)] = v * v

    pltpu.sync_copy(d2_v, dinv2_hbm.at[pl.ds(w * 320, 320)])

    pltpu.sync_copy(srcR_hbm.at[pl.ds(w * 40, 40)], srcn_v)
    pltpu.sync_copy(dstR_hbm.at[pl.ds(w * 40, 40)], dstn_v)
    pltpu.sync_copy(ewR_hbm.at[pl.ds(w * 40, 40)], ewn_v)

    @pl.loop(0, 40)
    def _(k):
        for q in range(8):
            sl = pl.ds(q * 16, 16)
            a = plsc.load_gather(dinv_v, [srcn_v[k, sl]])
            b = plsc.load_gather(dinv_v, [dstn_v[k, sl]])
            norm_v[k, sl] = a * ewn_v[k, sl] * b

    pltpu.sync_copy(norm_v, normR_hbm.at[pl.ds(w * 40, 40)])


def _make_agg(fh, edge_split):
    qg = fh // 16
    rpt = RPT // 2 if edge_split else RPT
    ng = rpt // 4

    @functools.partial(
        pl.kernel,
        out_type=jax.ShapeDtypeStruct((2 * N, fh), jnp.float32),
        mesh=_sc_mesh(),
        compiler_params=_SC_PARAMS,
        scratch_types=[
            pltpu.VMEM_SHARED((N, fh), jnp.float32),
            pltpu.VMEM((512,), jnp.int32),
            pltpu.VMEM((512,), jnp.int32),
            pltpu.VMEM((512,), jnp.float32),
            pltpu.VMEM((128, fh), jnp.float32),
            pltpu.VMEM((128, fh), jnp.float32),
            pltpu.SemaphoreType.DMA,
            pltpu.SemaphoreType.DMA,
            pltpu.SemaphoreType.DMA,
            pltpu.SemaphoreType.DMA,
            pltpu.SemaphoreType.DMA,
            pltpu.SemaphoreType.DMA,
        ],
    )
    def agg(h_hbm, srcF_hbm, dstF_hbm, normF_hbm, out_hbm,
            acc_sh, src_v, dst_v, norm_v, rows0, rows1,
            gs0, gs1, ss0, ss1, isem, isem2):
        c = lax.axis_index("c")
        s = lax.axis_index("s")
        row0 = (s * NCORE + c) * rpt if edge_split else s * rpt

        cn = jnp.full((16,), (0 if edge_split else N) * c, jnp.int32)

        @pl.loop(0, 128)
        def _(r):
            for q in range(qg):
                rows0[r, pl.ds(q * 16, 16)] = jnp.zeros((16,), jnp.float32)

        for m, sz in enumerate((128, 128, 128, 128, 112)):
            pltpu.sync_copy(
                rows0.at[pl.ds(0, sz)],
                acc_sh.at[pl.ds(pl.multiple_of(s * 624 + m * 128, 8), sz)])

        @pl.when(s == NSUB - 1)
        def _():
            pltpu.sync_copy(rows0.at[pl.ds(0, 16)],
                            acc_sh.at[pl.ds(NSUB * 624, 16)])

        plsc.subcore_barrier()

        def stage_src(g, start):
            base = pl.multiple_of((row0 + g * 4) * 128, 8)
            cp = (pltpu.async_copy if start else pltpu.make_async_copy)
            d = cp(srcF_hbm.at[pl.ds(base, 512)], src_v, isem2)
            if start:
                return
            d.wait()

            @plsc.parallel_loop(0, 32, unroll=4)
            def _(k):
                sl = pl.ds(k * 16, 16)
                src_v[sl] = src_v[sl] + cn

        def stage_dn(g, start):
            base = pl.multiple_of((row0 + g * 4) * 128, 8)
            cp = (pltpu.async_copy if start else pltpu.make_async_copy)
            d1 = cp(dstF_hbm.at[pl.ds(base, 512)], dst_v, isem)
            d2 = cp(normF_hbm.at[pl.ds(base, 512)], norm_v, isem)
            if not start:
                d1.wait()
                d2.wait()

        def scale(buf, jj):
            @plsc.parallel_loop(0, 128, unroll=8)
            def _(r):
                nv = plsc.load_gather(
                    norm_v, [jnp.full((16,), jj * 128 + r, jnp.int32)])
                for q in range(qg):
                    sl = pl.ds(q * 16, 16)
                    buf[r, sl] = buf[r, sl] * nv

        bufs = (rows0, rows1)
        sems = (gs0, gs1)
        ss = (ss0, ss1)

        def idx(jj):
            return pl.ds(jj * 128, 128)

        def wait_scatter(jj, p):
            pltpu.make_async_copy(
                bufs[p], acc_sh.at[dst_v.at[idx(jj)]], ss[p]).wait()

        stage_src(0, start=True)
        stage_src(0, start=False)
        pltpu.async_copy(h_hbm.at[src_v.at[idx(0)]], rows0, gs0)
        stage_dn(0, start=True)

        @pl.loop(0, ng)
        def _(t):
            for jj in range(4):
                p = jj % 2
                buf, sem = bufs[p], sems[p]
                pltpu.make_async_copy(
                    h_hbm.at[src_v.at[idx(jj)]], buf, sem).wait()
                if jj == 0:
                    stage_dn(t, start=False)
                if jj >= 1:
                    wait_scatter(jj - 1, 1 - p)
                if jj < 3:
                    pltpu.async_copy(h_hbm.at[src_v.at[idx(jj + 1)]],
                                     bufs[1 - p], sems[1 - p])
                if jj == 3:
                    @pl.when(t < ng - 1)
                    def _():
                        stage_src(t + 1, start=True)
                scale(buf, jj)
                pltpu.async_copy(buf, acc_sh.at[dst_v.at[idx(jj)]],
                                 ss[p], add=True)

            @pl.when(t < ng - 1)
            def _():
                stage_src(t + 1, start=False)
                pltpu.async_copy(h_hbm.at[src_v.at[idx(0)]], rows0, gs0)
                wait_scatter(3, 1)
                stage_dn(t + 1, start=True)

        wait_scatter(3, 1)
        plsc.subcore_barrier()

        pltpu.sync_copy(
            acc_sh.at[pl.ds(pl.multiple_of(s * 624, 8), 624)],
            out_hbm.at[pl.ds(pl.multiple_of(c * N + s * 624, 8), 624)])

        @pl.when(s == NSUB - 1)
        def _():
            pltpu.sync_copy(acc_sh.at[pl.ds(NSUB * 624, 16)],
                            out_hbm.at[pl.ds(c * N + NSUB * 624, 16)])

    return agg


_agg128 = _make_agg(128, edge_split=False)
_agg3 = _make_agg(64, edge_split=True)


def _mm_split(x, W):
    n, d = x.shape
    f = W.shape[1]
    fh = f // 2
    bn = 2000
    nb = n // bn

    def body(x_ref, w_ref, o_ref):
        o_ref[...] = jnp.dot(x_ref[...], w_ref[...],
                             preferred_element_type=jnp.float32)

    return pl.pallas_call(
        body,
        grid=(nb, 2),
        in_specs=[pl.BlockSpec((bn, d), lambda i, j: (i, 0)),
                  pl.BlockSpec((d, fh), lambda i, j: (0, j))],
        out_specs=pl.BlockSpec((bn, fh), lambda i, j: (j * nb + i, 0)),
        out_shape=jax.ShapeDtypeStruct((2 * n, fh), jnp.float32),
    )(x, W)


def _post(agg, h, dinv2, b):
    f = b.shape[0]
    fh = f // 2
    bn = 2000
    nb = N // bn

    def body(a0, a1, h0, h1, d2, b_ref, z_ref, st_ref):
        d2v = d2[...]
        z0 = a0[...] + h0[...] * d2v
        z1 = a1[...] + h1[...] * d2v
        z = jnp.concatenate([z0, z1], axis=1) + b_ref[...]
        z_ref[...] = z
        s0 = jnp.sum(z, axis=0, keepdims=True)
        s1 = jnp.sum(z * z, axis=0, keepdims=True)
        upd = jnp.concatenate([s0, s1, jnp.zeros((6, f), jnp.float32)], axis=0)

        @pl.when(pl.program_id(0) == 0)
        def _():
            st_ref[...] = jnp.zeros_like(st_ref)

        st_ref[...] += upd

    return pl.pallas_call(
        body,
        grid=(nb,),
        in_specs=[pl.BlockSpec((bn, fh), lambda i: (i, 0)),
                  pl.BlockSpec((bn, fh), lambda i: (nb + i, 0)),
                  pl.BlockSpec((bn, fh), lambda i: (i, 0)),
                  pl.BlockSpec((bn, fh), lambda i: (nb + i, 0)),
                  pl.BlockSpec((bn, 1), lambda i: (i, 0)),
                  pl.BlockSpec((1, f), lambda i: (0, 0))],
        out_specs=[pl.BlockSpec((bn, f), lambda i: (i, 0)),
                   pl.BlockSpec((8, f), lambda i: (0, 0))],
        out_shape=[jax.ShapeDtypeStruct((N, f), jnp.float32),
                   jax.ShapeDtypeStruct((8, f), jnp.float32)],
    )(agg, agg, h, h, dinv2, b.reshape(1, f))


def _bn_relu_mm(z, st, g, be, W, split):
    n, f = z.shape
    fo = W.shape[1]
    fh = fo // 2 if split else fo
    bn = 2000
    nb = n // bn

    def body(z_ref, st_ref, g_ref, be_ref, w_ref, o_ref):
        mu = st_ref[0:1, :] * (1.0 / n)
        ex2 = st_ref[1:2, :] * (1.0 / n)
        var = ex2 - mu * mu
        istd = lax.rsqrt(var + 1e-5)
        y = jnp.maximum((z_ref[...] - mu) * istd * g_ref[...] + be_ref[...],
                        0.0)
        o_ref[...] = jnp.dot(y, w_ref[...], preferred_element_type=jnp.float32)

    if split:
        grid = (nb, 2)
        w_spec = pl.BlockSpec((f, fh), lambda i, j: (0, j))
        o_spec = pl.BlockSpec((bn, fh), lambda i, j: (j * nb + i, 0))
        o_shape = jax.ShapeDtypeStruct((2 * n, fh), jnp.float32)
        bcast = lambda m: pl.BlockSpec(m, lambda i, j: (0, 0))
        z_spec = pl.BlockSpec((bn, f), lambda i, j: (i, 0))
    else:
        grid = (nb,)
        w_spec = pl.BlockSpec((f, fh), lambda i: (0, 0))
        o_spec = pl.BlockSpec((bn, fh), lambda i: (i, 0))
        o_shape = jax.ShapeDtypeStruct((n, fh), jnp.float32)
        bcast = lambda m: pl.BlockSpec(m, lambda i: (0, 0))
        z_spec = pl.BlockSpec((bn, f), lambda i: (i, 0))

    return pl.pallas_call(
        body,
        grid=grid,
        in_specs=[z_spec, bcast((8, f)), bcast((1, f)), bcast((1, f)), w_spec],
        out_specs=o_spec,
        out_shape=o_shape,
    )(z, st, g.reshape(1, f), be.reshape(1, f), W)


def _final(agg, h, dinv2, b):
    f = b.shape[0]
    bn = 2000
    nb = N // bn

    def body(a0, a1, h_ref, d2, b_ref, o_ref):
        z = a0[...] + a1[...] + h_ref[...] * d2[...] + b_ref[...]
        m = jnp.max(z, axis=1, keepdims=True)
        e = jnp.exp(z - m)
        lse = jnp.log(jnp.sum(e, axis=1, keepdims=True)) + m
        o_ref[...] = z - lse

    return pl.pallas_call(
        body,
        grid=(nb,),
        in_specs=[pl.BlockSpec((bn, f), lambda i: (i, 0)),
                  pl.BlockSpec((bn, f), lambda i: (nb + i, 0)),
                  pl.BlockSpec((bn, f), lambda i: (i, 0)),
                  pl.BlockSpec((bn, 1), lambda i: (i, 0)),
                  pl.BlockSpec((1, f), lambda i: (0, 0))],
        out_specs=pl.BlockSpec((bn, f), lambda i: (i, 0)),
        out_shape=jax.ShapeDtypeStruct((N, f), jnp.float32),
    )(agg, agg, h, dinv2, b.reshape(1, f))


def kernel(x, edge_index, edge_weight, W1, b1, g1, be1, W2, b2, g2, be2, W3, b3):
    src = edge_index[0]
    dst = edge_index[1]
    pad = EP - E
    srcR = jnp.concatenate(
        [src, jnp.zeros((pad,), jnp.int32)]).reshape(ER, 128)
    dstR = jnp.concatenate(
        [dst, jnp.zeros((pad,), jnp.int32)]).reshape(ER, 128)
    ewR = jnp.concatenate(
        [edge_weight, jnp.zeros((pad,), jnp.float32)]).reshape(ER, 128)

    normR, dinv2p = _sc_norm(srcR, dstR, ewR)
    dinv2 = dinv2p[:N].reshape(N, 1)
    srcF = srcR.reshape(EP)
    dstF = dstR.reshape(EP)
    normF = normR.reshape(EP)

    h1 = _mm_split(x, W1)
    a1 = _agg128(h1, srcF, dstF, normF)
    z1, st1 = _post(a1, h1, dinv2, b1)
    h2 = _bn_relu_mm(z1, st1, g1, be1, W2, split=True)
    a2 = _agg128(h2, srcF, dstF, normF)
    z2, st2 = _post(a2, h2, dinv2, b2)
    h3 = _bn_relu_mm(z2, st2, g2, be2, W3, split=False)
    a3 = _agg3(h3, srcF, dstF, normF)
    return _final(a3, h3, dinv2, b3)


# speedup vs baseline: 1.1496x; 1.0005x over previous
import functools

import jax
import jax.numpy as jnp
from jax import lax
from jax.experimental import pallas as pl
from jax.experimental.pallas import tpu as pltpu
from jax.experimental.pallas import tpu_sc as plsc

N = 10000
NP = 10240
E = 160000
EP = 163840
ER = EP // 128
RPT = ER // 16
NSUB = 16
NCORE = 2


def _sc_mesh():
    return plsc.VectorSubcoreMesh(core_axis_name="c", subcore_axis_name="s")


_SC_PARAMS = pltpu.CompilerParams(needs_layout_passes=False,
                                  use_tc_tiling_on_sc=False)


@functools.partial(
    pl.kernel,
    out_type=(
        jax.ShapeDtypeStruct((ER, 128), jnp.float32),
        jax.ShapeDtypeStruct((NP,), jnp.float32),
    ),
    mesh=_sc_mesh(),
    compiler_params=_SC_PARAMS,
    scratch_types=[
        pltpu.VMEM_SHARED((NP,), jnp.float32),
        pltpu.VMEM((RPT, 128), jnp.int32),
        pltpu.VMEM((RPT, 128), jnp.float32),
        pltpu.VMEM((NP,), jnp.float32),
        pltpu.VMEM((40, 128), jnp.int32),
        pltpu.VMEM((40, 128), jnp.int32),
        pltpu.VMEM((40, 128), jnp.float32),
        pltpu.VMEM((40, 128), jnp.float32),
        pltpu.VMEM((320,), jnp.float32),
        pltpu.VMEM((640,), jnp.float32),
    ],
)
def _sc_norm(srcR_hbm, dstR_hbm, ewR_hbm, normR_hbm, dinv2_hbm,
             deg_sh, dst_v, ew_v, dinv_v, srcn_v, dstn_v, ewn_v,
             norm_v, d2_v, zb):
    c = lax.axis_index("c")
    s = lax.axis_index("s")
    w = s * NCORE + c

    @pl.loop(0, 40)
    def _(k):
        zb[pl.ds(k * 16, 16)] = jnp.zeros((16,), jnp.float32)

    pltpu.sync_copy(zb, deg_sh.at[pl.ds(s * 640, 640)])
    plsc.subcore_barrier()

    pltpu.sync_copy(dstR_hbm.at[pl.ds(s * RPT, RPT)], dst_v)
    pltpu.sync_copy(ewR_hbm.at[pl.ds(s * RPT, RPT)], ew_v)

    @pl.loop(0, RPT)
    def _(k):
        pltpu.sync_copy(ew_v.at[k], deg_sh.at[dst_v.at[k]], add=True)

    plsc.subcore_barrier()

    pltpu.sync_copy(deg_sh, dinv_v)

    @plsc.parallel_loop(0, NP // 16, unroll=2)
    def _(k):
        sl = pl.ds(k * 16, 16)
        d = dinv_v[sl] + 1.0
        i = plsc.bitcast(d, jnp.int32)
        y = plsc.bitcast(jnp.full((16,), 0x5F3759DF, jnp.int32)
                         - lax.shift_right_logical(i, 1), jnp.float32)
        y = y * (1.5 - 0.5 * d * y * y)
        y = y * (1.5 - 0.5 * d * y * y)
        y = y * (1.5 - 0.5 * d * y * y)
        y = y * (1.5 - 0.5 * d * y * y)
        dinv_v[sl] = y

    @pl.loop(0, 20)
    def _(k):
        v = dinv_v[pl.ds(w * 320 + k * 16, 16)]
        d2_v[pl.ds(k * 16, 16)] = v * v

    pltpu.sync_copy(d2_v, dinv2_hbm.at[pl.ds(w * 320, 320)])

    pltpu.sync_copy(srcR_hbm.at[pl.ds(w * 40, 40)], srcn_v)
    pltpu.sync_copy(dstR_hbm.at[pl.ds(w * 40, 40)], dstn_v)
    pltpu.sync_copy(ewR_hbm.at[pl.ds(w * 40, 40)], ewn_v)

    @pl.loop(0, 40)
    def _(k):
        for q in range(8):
            sl = pl.ds(q * 16, 16)
            a = plsc.load_gather(dinv_v, [srcn_v[k, sl]])
            b = plsc.load_gather(dinv_v, [dstn_v[k, sl]])
            norm_v[k, sl] = a * ewn_v[k, sl] * b

    pltpu.sync_copy(norm_v, normR_hbm.at[pl.ds(w * 40, 40)])


def _make_agg(fh, edge_split):
    qg = fh // 16
    rpt = RPT // 2 if edge_split else RPT
    depth = 4 if edge_split else 2
    grp = 8 if edge_split else 4
    ng = rpt // grp

    @functools.partial(
        pl.kernel,
        out_type=jax.ShapeDtypeStruct((2 * N, fh), jnp.float32),
        mesh=_sc_mesh(),
        compiler_params=_SC_PARAMS,
        scratch_types=[
            pltpu.VMEM_SHARED((N, fh), jnp.float32),
            pltpu.VMEM((grp * 128,), jnp.int32),
            pltpu.VMEM((grp * 128,), jnp.int32),
            pltpu.VMEM((grp * 128,), jnp.float32),
        ] + [pltpu.VMEM((128, fh), jnp.float32)] * depth
          + [pltpu.SemaphoreType.DMA] * (2 * depth + 2),
    )
    def agg(h_hbm, srcF_hbm, dstF_hbm, normF_hbm, out_hbm,
            acc_sh, src_v, dst_v, norm_v, *bufs_and_sems):
        bufs = bufs_and_sems[:depth]
        sems = bufs_and_sems[depth:2 * depth]
        ss = bufs_and_sems[2 * depth:3 * depth]
        isem, isem2 = bufs_and_sems[3 * depth:]
        rows0 = bufs[0]
        c = lax.axis_index("c")
        s = lax.axis_index("s")
        row0 = (s * NCORE + c) * rpt if edge_split else s * rpt

        cn = jnp.full((16,), (0 if edge_split else N) * c, jnp.int32)

        @pl.loop(0, 128)
        def _(r):
            for q in range(qg):
                rows0[r, pl.ds(q * 16, 16)] = jnp.zeros((16,), jnp.float32)

        for m, sz in enumerate((128, 128, 128, 128, 112)):
            pltpu.sync_copy(
                rows0.at[pl.ds(0, sz)],
                acc_sh.at[pl.ds(pl.multiple_of(s * 624 + m * 128, 8), sz)])

        @pl.when(s == NSUB - 1)
        def _():
            pltpu.sync_copy(rows0.at[pl.ds(0, 16)],
                            acc_sh.at[pl.ds(NSUB * 624, 16)])

        plsc.subcore_barrier()

        def stage_src(g, start):
            base = pl.multiple_of((row0 + g * grp) * 128, 8)
            cp = (pltpu.async_copy if start else pltpu.make_async_copy)
            d = cp(srcF_hbm.at[pl.ds(base, grp * 128)], src_v, isem2)
            if start:
                return
            d.wait()

            @plsc.parallel_loop(0, grp * 8, unroll=4)
            def _(k):
                sl = pl.ds(k * 16, 16)
                src_v[sl] = src_v[sl] + cn

        def stage_dn(g, start):
            base = pl.multiple_of((row0 + g * grp) * 128, 8)
            cp = (pltpu.async_copy if start else pltpu.make_async_copy)
            d1 = cp(dstF_hbm.at[pl.ds(base, grp * 128)], dst_v, isem)
            d2 = cp(normF_hbm.at[pl.ds(base, grp * 128)], norm_v, isem)
            if not start:
                d1.wait()
                d2.wait()

        def scale(buf, jj):
            @plsc.parallel_loop(0, 128, unroll=8)
            def _(r):
                nv = plsc.load_gather(
                    norm_v, [jnp.full((16,), jj * 128 + r, jnp.int32)])
                for q in range(qg):
                    sl = pl.ds(q * 16, 16)
                    buf[r, sl] = buf[r, sl] * nv

        def idx(jj):
            return pl.ds(jj * 128, 128)

        def wait_scatter(jj, p):
            pltpu.make_async_copy(
                bufs[p], acc_sh.at[dst_v.at[idx(jj)]], ss[p]).wait()

        stage_src(0, start=True)
        stage_src(0, start=False)
        pltpu.async_copy(h_hbm.at[src_v.at[idx(0)]], bufs[0], sems[0])
        stage_dn(0, start=True)

        @pl.loop(0, ng)
        def _(t):
            for jj in range(grp):
                p = jj % depth
                buf, sem = bufs[p], sems[p]
                pltpu.make_async_copy(
                    h_hbm.at[src_v.at[idx(jj)]], buf, sem).wait()
                if jj == 0:
                    stage_dn(t, start=False)
                if jj + 1 - depth >= 0:
                    wait_scatter(jj + 1 - depth, (jj + 1) % depth)
                if jj < grp - 1:
                    pltpu.async_copy(h_hbm.at[src_v.at[idx(jj + 1)]],
                                     bufs[(jj + 1) % depth],
                                     sems[(jj + 1) % depth])
                if jj == grp - 1:
                    @pl.when(t < ng - 1)
                    def _():
                        stage_src(t + 1, start=True)
                scale(buf, jj)
                pltpu.async_copy(buf, acc_sh.at[dst_v.at[idx(jj)]],
                                 ss[p], add=True)

            @pl.when(t < ng - 1)
            def _():
                stage_src(t + 1, start=False)
                pltpu.async_copy(h_hbm.at[src_v.at[idx(0)]], bufs[0],
                                 sems[0])
                for d in range(grp - depth + 1, grp):
                    wait_scatter(d, d % depth)
                stage_dn(t + 1, start=True)

        for d in range(grp - depth + 1, grp):
            wait_scatter(d, d % depth)
        plsc.subcore_barrier()

        pltpu.sync_copy(
            acc_sh.at[pl.ds(pl.multiple_of(s * 624, 8), 624)],
            out_hbm.at[pl.ds(pl.multiple_of(c * N + s * 624, 8), 624)])

        @pl.when(s == NSUB - 1)
        def _():
            pltpu.sync_copy(acc_sh.at[pl.ds(NSUB * 624, 16)],
                            out_hbm.at[pl.ds(c * N + NSUB * 624, 16)])

    return agg


_agg128 = _make_agg(128, edge_split=False)
_agg3 = _make_agg(64, edge_split=True)


def _mm_split(x, W):
    n, d = x.shape
    f = W.shape[1]
    fh = f // 2
    bn = 2000
    nb = n // bn

    def body(x_ref, w_ref, o_ref):
        o_ref[...] = jnp.dot(x_ref[...], w_ref[...],
                             preferred_element_type=jnp.float32)

    return pl.pallas_call(
        body,
        grid=(nb, 2),
        in_specs=[pl.BlockSpec((bn, d), lambda i, j: (i, 0)),
                  pl.BlockSpec((d, fh), lambda i, j: (0, j))],
        out_specs=pl.BlockSpec((bn, fh), lambda i, j: (j * nb + i, 0)),
        out_shape=jax.ShapeDtypeStruct((2 * n, fh), jnp.float32),
    )(x, W)


def _post(agg, h, dinv2, b):
    f = b.shape[0]
    fh = f // 2
    bn = 2000
    nb = N // bn

    def body(a0, a1, h0, h1, d2, b_ref, z_ref, st_ref):
        d2v = d2[...]
        z0 = a0[...] + h0[...] * d2v
        z1 = a1[...] + h1[...] * d2v
        z = jnp.concatenate([z0, z1], axis=1) + b_ref[...]
        z_ref[...] = z
        s0 = jnp.sum(z, axis=0, keepdims=True)
        s1 = jnp.sum(z * z, axis=0, keepdims=True)
        upd = jnp.concatenate([s0, s1, jnp.zeros((6, f), jnp.float32)], axis=0)

        @pl.when(pl.program_id(0) == 0)
        def _():
            st_ref[...] = jnp.zeros_like(st_ref)

        st_ref[...] += upd

    return pl.pallas_call(
        body,
        grid=(nb,),
        in_specs=[pl.BlockSpec((bn, fh), lambda i: (i, 0)),
                  pl.BlockSpec((bn, fh), lambda i: (nb + i, 0)),
                  pl.BlockSpec((bn, fh), lambda i: (i, 0)),
                  pl.BlockSpec((bn, fh), lambda i: (nb + i, 0)),
                  pl.BlockSpec((bn, 1), lambda i: (i, 0)),
                  pl.BlockSpec((1, f), lambda i: (0, 0))],
        out_specs=[pl.BlockSpec((bn, f), lambda i: (i, 0)),
                   pl.BlockSpec((8, f), lambda i: (0, 0))],
        out_shape=[jax.ShapeDtypeStruct((N, f), jnp.float32),
                   jax.ShapeDtypeStruct((8, f), jnp.float32)],
    )(agg, agg, h, h, dinv2, b.reshape(1, f))


def _bn_relu_mm(z, st, g, be, W, split):
    n, f = z.shape
    fo = W.shape[1]
    fh = fo // 2 if split else fo
    bn = 2000
    nb = n // bn

    def body(z_ref, st_ref, g_ref, be_ref, w_ref, o_ref):
        mu = st_ref[0:1, :] * (1.0 / n)
        ex2 = st_ref[1:2, :] * (1.0 / n)
        var = ex2 - mu * mu
        istd = lax.rsqrt(var + 1e-5)
        y = jnp.maximum((z_ref[...] - mu) * istd * g_ref[...] + be_ref[...],
                        0.0)
        o_ref[...] = jnp.dot(y, w_ref[...], preferred_element_type=jnp.float32)

    if split:
        grid = (nb, 2)
        w_spec = pl.BlockSpec((f, fh), lambda i, j: (0, j))
        o_spec = pl.BlockSpec((bn, fh), lambda i, j: (j * nb + i, 0))
        o_shape = jax.ShapeDtypeStruct((2 * n, fh), jnp.float32)
        bcast = lambda m: pl.BlockSpec(m, lambda i, j: (0, 0))
        z_spec = pl.BlockSpec((bn, f), lambda i, j: (i, 0))
    else:
        grid = (nb,)
        w_spec = pl.BlockSpec((f, fh), lambda i: (0, 0))
        o_spec = pl.BlockSpec((bn, fh), lambda i: (i, 0))
        o_shape = jax.ShapeDtypeStruct((n, fh), jnp.float32)
        bcast = lambda m: pl.BlockSpec(m, lambda i: (0, 0))
        z_spec = pl.BlockSpec((bn, f), lambda i: (i, 0))

    return pl.pallas_call(
        body,
        grid=grid,
        in_specs=[z_spec, bcast((8, f)), bcast((1, f)), bcast((1, f)), w_spec],
        out_specs=o_spec,
        out_shape=o_shape,
    )(z, st, g.reshape(1, f), be.reshape(1, f), W)


def _final(agg, h, dinv2, b):
    f = b.shape[0]
    bn = 2000
    nb = N // bn

    def body(a0, a1, h_ref, d2, b_ref, o_ref):
        z = a0[...] + a1[...] + h_ref[...] * d2[...] + b_ref[...]
        m = jnp.max(z, axis=1, keepdims=True)
        e = jnp.exp(z - m)
        lse = jnp.log(jnp.sum(e, axis=1, keepdims=True)) + m
        o_ref[...] = z - lse

    return pl.pallas_call(
        body,
        grid=(nb,),
        in_specs=[pl.BlockSpec((bn, f), lambda i: (i, 0)),
                  pl.BlockSpec((bn, f), lambda i: (nb + i, 0)),
                  pl.BlockSpec((bn, f), lambda i: (i, 0)),
                  pl.BlockSpec((bn, 1), lambda i: (i, 0)),
                  pl.BlockSpec((1, f), lambda i: (0, 0))],
        out_specs=pl.BlockSpec((bn, f), lambda i: (i, 0)),
        out_shape=jax.ShapeDtypeStruct((N, f), jnp.float32),
    )(agg, agg, h, dinv2, b.reshape(1, f))


def kernel(x, edge_index, edge_weight, W1, b1, g1, be1, W2, b2, g2, be2, W3, b3):
    src = edge_index[0]
    dst = edge_index[1]
    pad = EP - E
    srcR = jnp.concatenate(
        [src, jnp.zeros((pad,), jnp.int32)]).reshape(ER, 128)
    dstR = jnp.concatenate(
        [dst, jnp.zeros((pad,), jnp.int32)]).reshape(ER, 128)
    ewR = jnp.concatenate(
        [edge_weight, jnp.zeros((pad,), jnp.float32)]).reshape(ER, 128)

    normR, dinv2p = _sc_norm(srcR, dstR, ewR)
    dinv2 = dinv2p[:N].reshape(N, 1)
    srcF = srcR.reshape(EP)
    dstF = dstR.reshape(EP)
    normF = normR.reshape(EP)

    h1 = _mm_split(x, W1)
    a1 = _agg128(h1, srcF, dstF, normF)
    z1, st1 = _post(a1, h1, dinv2, b1)
    h2 = _bn_relu_mm(z1, st1, g1, be1, W2, split=True)
    a2 = _agg128(h2, srcF, dstF, normF)
    z2, st2 = _post(a2, h2, dinv2, b2)
    h3 = _bn_relu_mm(z2, st2, g2, be2, W3, split=False)
    a3 = _agg3(h3, srcF, dstF, normF)
    return _final(a3, h3, dinv2, b3)


# speedup vs baseline: 1.1520x; 1.0020x over previous
import functools

import jax
import jax.numpy as jnp
from jax import lax
from jax.experimental import pallas as pl
from jax.experimental.pallas import tpu as pltpu
from jax.experimental.pallas import tpu_sc as plsc

N = 10000
NP = 10240
E = 160000
EP = 163840
ER = EP // 128
RPT = ER // 16
NSUB = 16
NCORE = 2


def _sc_mesh():
    return plsc.VectorSubcoreMesh(core_axis_name="c", subcore_axis_name="s")


_SC_PARAMS = pltpu.CompilerParams(needs_layout_passes=False,
                                  use_tc_tiling_on_sc=False)


@functools.partial(
    pl.kernel,
    out_type=(
        jax.ShapeDtypeStruct((ER, 128), jnp.float32),
        jax.ShapeDtypeStruct((NP,), jnp.float32),
    ),
    mesh=_sc_mesh(),
    compiler_params=_SC_PARAMS,
    scratch_types=[
        pltpu.VMEM_SHARED((NP,), jnp.float32),
        pltpu.VMEM((RPT, 128), jnp.int32),
        pltpu.VMEM((RPT, 128), jnp.float32),
        pltpu.VMEM((NP,), jnp.float32),
        pltpu.VMEM((40, 128), jnp.int32),
        pltpu.VMEM((40, 128), jnp.int32),
        pltpu.VMEM((40, 128), jnp.float32),
        pltpu.VMEM((40, 128), jnp.float32),
        pltpu.VMEM((320,), jnp.float32),
        pltpu.VMEM((640,), jnp.float32),
    ],
)
def _sc_norm(srcR_hbm, dstR_hbm, ewR_hbm, normR_hbm, dinv2_hbm,
             deg_sh, dst_v, ew_v, dinv_v, srcn_v, dstn_v, ewn_v,
             norm_v, d2_v, zb):
    c = lax.axis_index("c")
    s = lax.axis_index("s")
    w = s * NCORE + c

    @pl.loop(0, 40)
    def _(k):
        zb[pl.ds(k * 16, 16)] = jnp.zeros((16,), jnp.float32)

    pltpu.sync_copy(zb, deg_sh.at[pl.ds(s * 640, 640)])
    plsc.subcore_barrier()

    pltpu.sync_copy(dstR_hbm.at[pl.ds(s * RPT, RPT)], dst_v)
    pltpu.sync_copy(ewR_hbm.at[pl.ds(s * RPT, RPT)], ew_v)

    @pl.loop(0, RPT)
    def _(k):
        pltpu.sync_copy(ew_v.at[k], deg_sh.at[dst_v.at[k]], add=True)

    plsc.subcore_barrier()

    pltpu.sync_copy(deg_sh, dinv_v)

    @plsc.parallel_loop(0, NP // 16, unroll=2)
    def _(k):
        sl = pl.ds(k * 16, 16)
        d = dinv_v[sl] + 1.0
        i = plsc.bitcast(d, jnp.int32)
        y = plsc.bitcast(jnp.full((16,), 0x5F3759DF, jnp.int32)
                         - lax.shift_right_logical(i, 1), jnp.float32)
        y = y * (1.5 - 0.5 * d * y * y)
        y = y * (1.5 - 0.5 * d * y * y)
        y = y * (1.5 - 0.5 * d * y * y)
        y = y * (1.5 - 0.5 * d * y * y)
        dinv_v[sl] = y

    @pl.loop(0, 20)
    def _(k):
        v = dinv_v[pl.ds(w * 320 + k * 16, 16)]
        d2_v[pl.ds(k * 16, 16)] = v * v

    pltpu.sync_copy(d2_v, dinv2_hbm.at[pl.ds(w * 320, 320)])

    pltpu.sync_copy(srcR_hbm.at[pl.ds(w * 40, 40)], srcn_v)
    pltpu.sync_copy(dstR_hbm.at[pl.ds(w * 40, 40)], dstn_v)
    pltpu.sync_copy(ewR_hbm.at[pl.ds(w * 40, 40)], ewn_v)

    @pl.loop(0, 40)
    def _(k):
        for q in range(8):
            sl = pl.ds(q * 16, 16)
            a = plsc.load_gather(dinv_v, [srcn_v[k, sl]])
            b = plsc.load_gather(dinv_v, [dstn_v[k, sl]])
            norm_v[k, sl] = a * ewn_v[k, sl] * b

    pltpu.sync_copy(norm_v, normR_hbm.at[pl.ds(w * 40, 40)])


def _make_agg(fh, edge_split):
    qg = fh // 16
    rpt = RPT // 2 if edge_split else RPT
    ng = rpt // 4

    @functools.partial(
        pl.kernel,
        out_type=jax.ShapeDtypeStruct((2 * N, fh), jnp.float32),
        mesh=_sc_mesh(),
        compiler_params=_SC_PARAMS,
        scratch_types=[
            pltpu.VMEM_SHARED((N, fh), jnp.float32),
            pltpu.VMEM((512,), jnp.int32),
            pltpu.VMEM((512,), jnp.int32),
            pltpu.VMEM((512,), jnp.float32),
            pltpu.VMEM((128, fh), jnp.float32),
            pltpu.VMEM((128, fh), jnp.float32),
            pltpu.SemaphoreType.DMA,
            pltpu.SemaphoreType.DMA,
            pltpu.SemaphoreType.DMA,
            pltpu.SemaphoreType.DMA,
            pltpu.SemaphoreType.DMA,
            pltpu.SemaphoreType.DMA,
        ],
    )
    def agg(h_hbm, srcF_hbm, dstF_hbm, normF_hbm, out_hbm,
            acc_sh, src_v, dst_v, norm_v, rows0, rows1,
            gs0, gs1, ss0, ss1, isem, isem2):
        c = lax.axis_index("c")
        s = lax.axis_index("s")
        row0 = (s * NCORE + c) * rpt if edge_split else s * rpt

        cn = jnp.full((16,), (0 if edge_split else N) * c, jnp.int32)

        @pl.loop(0, 128)
        def _(r):
            for q in range(qg):
                rows0[r, pl.ds(q * 16, 16)] = jnp.zeros((16,), jnp.float32)

        for m, sz in enumerate((128, 128, 128, 128, 112)):
            pltpu.sync_copy(
                rows0.at[pl.ds(0, sz)],
                acc_sh.at[pl.ds(pl.multiple_of(s * 624 + m * 128, 8), sz)])

        @pl.when(s == NSUB - 1)
        def _():
            pltpu.sync_copy(rows0.at[pl.ds(0, 16)],
                            acc_sh.at[pl.ds(NSUB * 624, 16)])

        plsc.subcore_barrier()

        def stage_src(g, start):
            base = pl.multiple_of((row0 + g * 4) * 128, 8)
            cp = (pltpu.async_copy if start else pltpu.make_async_copy)
            d = cp(srcF_hbm.at[pl.ds(base, 512)], src_v, isem2)
            if start:
                return
            d.wait()

            @plsc.parallel_loop(0, 32, unroll=4)
            def _(k):
                sl = pl.ds(k * 16, 16)
                src_v[sl] = src_v[sl] + cn

        def stage_dn(g, start):
            base = pl.multiple_of((row0 + g * 4) * 128, 8)
            cp = (pltpu.async_copy if start else pltpu.make_async_copy)
            d1 = cp(dstF_hbm.at[pl.ds(base, 512)], dst_v, isem)
            d2 = cp(normF_hbm.at[pl.ds(base, 512)], norm_v, isem)
            if not start:
                d1.wait()
                d2.wait()

        def scale(buf, jj):
            @plsc.parallel_loop(0, 128, unroll=8)
            def _(r):
                nv = plsc.load_gather(
                    norm_v, [jnp.full((16,), jj * 128 + r, jnp.int32)])
                for q in range(qg):
                    sl = pl.ds(q * 16, 16)
                    buf[r, sl] = buf[r, sl] * nv

        bufs = (rows0, rows1)
        sems = (gs0, gs1)
        ss = (ss0, ss1)

        def idx(jj):
            return pl.ds(jj * 128, 128)

        def wait_scatter(jj, p):
            pltpu.make_async_copy(
                bufs[p], acc_sh.at[dst_v.at[idx(jj)]], ss[p]).wait()

        stage_src(0, start=True)
        stage_src(0, start=False)
        pltpu.async_copy(h_hbm.at[src_v.at[idx(0)]], rows0, gs0)
        stage_dn(0, start=True)

        @pl.loop(0, ng)
        def _(t):
            for jj in range(4):
                p = jj % 2
                buf, sem = bufs[p], sems[p]
                pltpu.make_async_copy(
                    h_hbm.at[src_v.at[idx(jj)]], buf, sem).wait()
                if jj == 0:
                    stage_dn(t, start=False)
                if jj >= 1:
                    wait_scatter(jj - 1, 1 - p)
                if jj < 3:
                    pltpu.async_copy(h_hbm.at[src_v.at[idx(jj + 1)]],
                                     bufs[1 - p], sems[1 - p])
                if jj == 3:
                    @pl.when(t < ng - 1)
                    def _():
                        stage_src(t + 1, start=True)
                scale(buf, jj)
                pltpu.async_copy(buf, acc_sh.at[dst_v.at[idx(jj)]],
                                 ss[p], add=True)

            @pl.when(t < ng - 1)
            def _():
                stage_src(t + 1, start=False)
                pltpu.async_copy(h_hbm.at[src_v.at[idx(0)]], rows0, gs0)
                wait_scatter(3, 1)
                stage_dn(t + 1, start=True)

        wait_scatter(3, 1)
        plsc.subcore_barrier()

        pltpu.sync_copy(
            acc_sh.at[pl.ds(pl.multiple_of(s * 624, 8), 624)],
            out_hbm.at[pl.ds(pl.multiple_of(c * N + s * 624, 8), 624)])

        @pl.when(s == NSUB - 1)
        def _():
            pltpu.sync_copy(acc_sh.at[pl.ds(NSUB * 624, 16)],
                            out_hbm.at[pl.ds(c * N + NSUB * 624, 16)])

    return agg


_agg128 = _make_agg(128, edge_split=False)
_agg3 = _make_agg(64, edge_split=True)


def _mm_split(x, W):
    n, d = x.shape
    f = W.shape[1]
    fh = f // 2
    bn = 2000
    nb = n // bn

    def body(x_ref, w_ref, o_ref):
        o_ref[...] = jnp.dot(x_ref[...], w_ref[...],
                             preferred_element_type=jnp.float32)

    return pl.pallas_call(
        body,
        grid=(nb, 2),
        in_specs=[pl.BlockSpec((bn, d), lambda i, j: (i, 0)),
                  pl.BlockSpec((d, fh), lambda i, j: (0, j))],
        out_specs=pl.BlockSpec((bn, fh), lambda i, j: (j * nb + i, 0)),
        out_shape=jax.ShapeDtypeStruct((2 * n, fh), jnp.float32),
    )(x, W)


def _post(agg, h, dinv2, b):
    f = b.shape[0]
    fh = f // 2
    bn = 2000
    nb = N // bn

    def body(a0, a1, h0, h1, d2, b_ref, z_ref, st_ref):
        d2v = d2[...]
        z0 = a0[...] + h0[...] * d2v
        z1 = a1[...] + h1[...] * d2v
        z = jnp.concatenate([z0, z1], axis=1) + b_ref[...]
        z_ref[...] = z
        s0 = jnp.sum(z, axis=0, keepdims=True)
        s1 = jnp.sum(z * z, axis=0, keepdims=True)
        upd = jnp.concatenate([s0, s1, jnp.zeros((6, f), jnp.float32)], axis=0)

        @pl.when(pl.program_id(0) == 0)
        def _():
            st_ref[...] = jnp.zeros_like(st_ref)

        st_ref[...] += upd

    return pl.pallas_call(
        body,
        grid=(nb,),
        in_specs=[pl.BlockSpec((bn, fh), lambda i: (i, 0)),
                  pl.BlockSpec((bn, fh), lambda i: (nb + i, 0)),
                  pl.BlockSpec((bn, fh), lambda i: (i, 0)),
                  pl.BlockSpec((bn, fh), lambda i: (nb + i, 0)),
                  pl.BlockSpec((bn, 1), lambda i: (i, 0)),
                  pl.BlockSpec((1, f), lambda i: (0, 0))],
        out_specs=[pl.BlockSpec((bn, f), lambda i: (i, 0)),
                   pl.BlockSpec((8, f), lambda i: (0, 0))],
        out_shape=[jax.ShapeDtypeStruct((N, f), jnp.float32),
                   jax.ShapeDtypeStruct((8, f), jnp.float32)],
    )(agg, agg, h, h, dinv2, b.reshape(1, f))


def _bn_relu_mm(z, st, g, be, W, split):
    n, f = z.shape
    fo = W.shape[1]
    fh = fo // 2 if split else fo
    bn = 2000
    nb = n // bn

    def body(z_ref, st_ref, g_ref, be_ref, w_ref, o_ref):
        mu = st_ref[0:1, :] * (1.0 / n)
        ex2 = st_ref[1:2, :] * (1.0 / n)
        var = ex2 - mu * mu
        istd = lax.rsqrt(var + 1e-5)
        y = jnp.maximum((z_ref[...] - mu) * istd * g_ref[...] + be_ref[...],
                        0.0)
        o_ref[...] = jnp.dot(y, w_ref[...], preferred_element_type=jnp.float32)

    if split:
        grid = (nb, 2)
        w_spec = pl.BlockSpec((f, fh), lambda i, j: (0, j))
        o_spec = pl.BlockSpec((bn, fh), lambda i, j: (j * nb + i, 0))
        o_shape = jax.ShapeDtypeStruct((2 * n, fh), jnp.float32)
        bcast = lambda m: pl.BlockSpec(m, lambda i, j: (0, 0))
        z_spec = pl.BlockSpec((bn, f), lambda i, j: (i, 0))
    else:
        grid = (nb,)
        w_spec = pl.BlockSpec((f, fh), lambda i: (0, 0))
        o_spec = pl.BlockSpec((bn, fh), lambda i: (i, 0))
        o_shape = jax.ShapeDtypeStruct((n, fh), jnp.float32)
        bcast = lambda m: pl.BlockSpec(m, lambda i: (0, 0))
        z_spec = pl.BlockSpec((bn, f), lambda i: (i, 0))

    return pl.pallas_call(
        body,
        grid=grid,
        in_specs=[z_spec, bcast((8, f)), bcast((1, f)), bcast((1, f)), w_spec],
        out_specs=o_spec,
        out_shape=o_shape,
    )(z, st, g.reshape(1, f), be.reshape(1, f), W)


def _final(agg, h, dinv2, b):
    f = b.shape[0]
    bn = 2000
    nb = N // bn

    def body(a0, a1, h_ref, d2, b_ref, o_ref):
        z = a0[...] + a1[...] + h_ref[...] * d2[...] + b_ref[...]
        m = jnp.max(z, axis=1, keepdims=True)
        e = jnp.exp(z - m)
        lse = jnp.log(jnp.sum(e, axis=1, keepdims=True)) + m
        o_ref[...] = z - lse

    return pl.pallas_call(
        body,
        grid=(nb,),
        in_specs=[pl.BlockSpec((bn, f), lambda i: (i, 0)),
                  pl.BlockSpec((bn, f), lambda i: (nb + i, 0)),
                  pl.BlockSpec((bn, f), lambda i: (i, 0)),
                  pl.BlockSpec((bn, 1), lambda i: (i, 0)),
                  pl.BlockSpec((1, f), lambda i: (0, 0))],
        out_specs=pl.BlockSpec((bn, f), lambda i: (i, 0)),
        out_shape=jax.ShapeDtypeStruct((N, f), jnp.float32),
    )(agg, agg, h, dinv2, b.reshape(1, f))


def kernel(x, edge_index, edge_weight, W1, b1, g1, be1, W2, b2, g2, be2, W3, b3):
    src = edge_index[0]
    dst = edge_index[1]
    pad = EP - E
    srcR = jnp.concatenate(
        [src, jnp.zeros((pad,), jnp.int32)]).reshape(ER, 128)
    dstR = jnp.concatenate(
        [dst, jnp.zeros((pad,), jnp.int32)]).reshape(ER, 128)
    ewR = jnp.concatenate(
        [edge_weight, jnp.zeros((pad,), jnp.float32)]).reshape(ER, 128)

    normR, dinv2p = _sc_norm(srcR, dstR, ewR)
    dinv2 = dinv2p[:N].reshape(N, 1)
    srcF = srcR.reshape(EP)
    dstF = dstR.reshape(EP)
    normF = normR.reshape(EP)

    h1 = _mm_split(x, W1)
    a1 = _agg128(h1, srcF, dstF, normF)
    z1, st1 = _post(a1, h1, dinv2, b1)
    h2 = _bn_relu_mm(z1, st1, g1, be1, W2, split=True)
    a2 = _agg128(h2, srcF, dstF, normF)
    z2, st2 = _post(a2, h2, dinv2, b2)
    h3 = _bn_relu_mm(z2, st2, g2, be2, W3, split=False)
    a3 = _agg3(h3, srcF, dstF, normF)
    return _final(a3, h3, dinv2, b3)
